# Initial kernel scaffold; baseline (speedup 1.0000x reference)
import jax
import jax.numpy as jnp
from jax.experimental import pallas as pl


def kernel(points1, points2, features1, features2, W1, b1, gamma1, beta1, W2, b2, gamma2, beta2):
    raise NotImplementedError("write your pallas kernel here")


# speedup vs baseline: 11.4323x; 11.4323x over previous
import functools

import jax
import jax.numpy as jnp
from jax import lax
from jax.experimental import pallas as pl
from jax.experimental.pallas import tpu as pltpu
from jax.experimental.pallas import tpu_sc as plsc

B, N1, N2 = 16, 1024, 4096
C1, C2 = 512, 256
O1, O2 = 512, 256
NQ = B * N2
EPS = 1e-3

QT = 256
RT = 512
NT2 = 256
NW = 32
QPW = NQ // NW
QCH = 32


def _gtable_body(f1_ref, w_ref, g_ref):
    g_ref[...] = lax.dot_general(
        f1_ref[0], w_ref[...], (((0,), (0,)), ((), ())),
        preferred_element_type=jnp.float32)


def _nn3_body(p2_ref, p1_ref, i0_ref, i1_ref, i2_ref, wexp_ref):
    b = pl.program_id(0)
    q = p2_ref[0]
    kpts = p1_ref[0]
    d2 = jnp.zeros((QT, N1), jnp.float32)
    for j in range(3):
        diff = q[:, j:j + 1] - kpts[j:j + 1, :]
        d2 = d2 + diff * diff
    lane = lax.broadcasted_iota(jnp.int32, (QT, N1), 1)
    cur = d2
    mins, args = [], []
    for _ in range(3):
        m = jnp.min(cur, axis=1, keepdims=True)
        am = jnp.min(jnp.where(cur == m, lane, jnp.int32(2 ** 30)),
                     axis=1, keepdims=True)
        cur = jnp.where(lane == am, jnp.float32(jnp.inf), cur)
        mins.append(m)
        args.append(am)
    invs = []
    for m in mins:
        d = jnp.sqrt(jnp.maximum(m, 0.0))
        dd = d * d
        dd = jnp.where(dd < 1e-10, jnp.float32(1e-10), dd)
        invs.append(1.0 / dd)
    norm = (invs[0] + invs[1]) + invs[2]
    base = b * N1
    i0_ref[0] = args[0] + base
    i1_ref[0] = args[1] + base
    i2_ref[0] = args[2] + base
    wexp_ref[0] = jnp.concatenate(
        [jnp.broadcast_to(iv / norm, (QT, 16)) for iv in invs], axis=1)


def _interp_body(g_hbm, i0_hbm, i1_hbm, i2_hbm, wexp_hbm,
                 out_hbm, i0v, i1v, i2v, wv, r0, r1, r2, ov, sem):
    cid = lax.axis_index("c")
    sid = lax.axis_index("s")
    wid = sid * 2 + cid
    base = wid * QPW
    pltpu.sync_copy(i0_hbm.at[pl.ds(base, QPW)], i0v)
    pltpu.sync_copy(i1_hbm.at[pl.ds(base, QPW)], i1v)
    pltpu.sync_copy(i2_hbm.at[pl.ds(base, QPW)], i2v)

    def chunk(ci, carry):
        cb = ci * QCH
        c0 = pltpu.async_copy(g_hbm.at[i0v.at[pl.ds(cb, QCH)]], r0, sem)
        c1 = pltpu.async_copy(g_hbm.at[i1v.at[pl.ds(cb, QCH)]], r1, sem)
        c2 = pltpu.async_copy(g_hbm.at[i2v.at[pl.ds(cb, QCH)]], r2, sem)
        pltpu.sync_copy(wexp_hbm.at[pl.ds((base + cb) * 48, QCH * 48)], wv)
        c0.wait()
        c1.wait()
        c2.wait()

        def qloop(qi, carry2):
            s0 = wv[pl.ds(qi * 48, 16)]
            s1 = wv[pl.ds(qi * 48 + 16, 16)]
            s2 = wv[pl.ds(qi * 48 + 32, 16)]
            for ch in range(O1 // 16):
                sl = pl.ds(ch * 16, 16)
                ov[qi, sl] = (s0 * r0[qi, sl] + s1 * r1[qi, sl]) \
                    + s2 * r2[qi, sl]
            return carry2

        lax.fori_loop(0, QCH, qloop, 0)
        pltpu.sync_copy(ov, out_hbm.at[pl.ds(base + cb, QCH)])
        return carry

    lax.fori_loop(0, QPW // QCH, chunk, 0)


def _mlp1_body(y1a_ref, f2_ref, w1b_ref, b1_ref, y_ref, s_ref, q_ref):
    y = y1a_ref[...] + lax.dot_general(
        f2_ref[0], w1b_ref[...], (((0,), (0,)), ((), ())),
        preferred_element_type=jnp.float32)
    y = y + b1_ref[...]
    y_ref[...] = y

    @pl.when((pl.program_id(0) == 0) & (pl.program_id(1) == 0))
    def _():
        s_ref[...] = jnp.zeros_like(s_ref)
        q_ref[...] = jnp.zeros_like(q_ref)

    s_ref[...] += jnp.sum(y, axis=0, keepdims=True)
    q_ref[...] += jnp.sum(y * y, axis=0, keepdims=True)


def _mlp2_body(y1_ref, a1_ref, c1_ref, w2_ref, b2_ref, y2_ref, s_ref, q_ref):
    h = jnp.maximum(y1_ref[...] * a1_ref[...] + c1_ref[...], 0.0)
    y = jnp.dot(h, w2_ref[...], preferred_element_type=jnp.float32)
    y = y + b2_ref[...]
    y2_ref[...] = y

    @pl.when(pl.program_id(0) == 0)
    def _():
        s_ref[...] = jnp.zeros_like(s_ref)
        q_ref[...] = jnp.zeros_like(q_ref)

    s_ref[...] += jnp.sum(y, axis=0, keepdims=True)
    q_ref[...] += jnp.sum(y * y, axis=0, keepdims=True)


def _fin_body(y2_ref, a2_ref, c2_ref, o_ref):
    h = jnp.maximum(y2_ref[...] * a2_ref[...] + c2_ref[...], 0.0)
    o_ref[0] = h.T


def kernel(points1, points2, features1, features2, W1, b1, gamma1, beta1,
           W2, b2, gamma2, beta2):
    p2t = jnp.transpose(points2, (0, 2, 1))
    w1aT = jnp.transpose(W1[:, :C1])
    w1bT = jnp.transpose(W1[:, C1:])
    w2T = jnp.transpose(W2)
    b1r = b1.reshape(1, O1)

    g = pl.pallas_call(
        _gtable_body,
        grid=(B,),
        in_specs=[
            pl.BlockSpec((1, C1, N1), lambda bb: (bb, 0, 0)),
            pl.BlockSpec((C1, O1), lambda bb: (0, 0)),
        ],
        out_specs=pl.BlockSpec((N1, O1), lambda bb: (bb, 0)),
        out_shape=jax.ShapeDtypeStruct((B * N1, O1), jnp.float32),
    )(features1, w1aT)

    nn = pl.pallas_call(
        _nn3_body,
        grid=(B, N2 // QT),
        in_specs=[
            pl.BlockSpec((1, QT, 3), lambda bb, t: (bb, t, 0)),
            pl.BlockSpec((1, 3, N1), lambda bb, t: (bb, 0, 0)),
        ],
        out_specs=(
            [pl.BlockSpec((1, QT, 1), lambda bb, t: (bb, t, 0))] * 3
            + [pl.BlockSpec((1, QT, 48), lambda bb, t: (bb, t, 0))]),
        out_shape=(
            [jax.ShapeDtypeStruct((B, N2, 1), jnp.int32)] * 3
            + [jax.ShapeDtypeStruct((B, N2, 48), jnp.float32)]),
    )(p2t, points1)
    i0, i1, i2 = (x.reshape(NQ) for x in nn[:3])
    wexp = nn[3].reshape(NQ * 48)

    mesh = plsc.VectorSubcoreMesh(core_axis_name="c", subcore_axis_name="s")
    interp = functools.partial(
        pl.kernel,
        mesh=mesh,
        out_type=jax.ShapeDtypeStruct((NQ, O1), jnp.float32),
        scratch_types=[
            pltpu.VMEM((QPW,), jnp.int32),
            pltpu.VMEM((QPW,), jnp.int32),
            pltpu.VMEM((QPW,), jnp.int32),
            pltpu.VMEM((QCH * 48,), jnp.float32),
            pltpu.VMEM((QCH, O1), jnp.float32),
            pltpu.VMEM((QCH, O1), jnp.float32),
            pltpu.VMEM((QCH, O1), jnp.float32),
            pltpu.VMEM((QCH, O1), jnp.float32),
            pltpu.SemaphoreType.DMA,
        ],
    )(_interp_body)
    y1a = interp(g, i0, i1, i2, wexp)

    y1raw, s1, q1 = pl.pallas_call(
        _mlp1_body,
        grid=(B, N2 // RT),
        in_specs=[
            pl.BlockSpec((RT, O1), lambda bb, t: (bb * (N2 // RT) + t, 0)),
            pl.BlockSpec((1, C2, RT), lambda bb, t: (bb, 0, t)),
            pl.BlockSpec((C2, O1), lambda bb, t: (0, 0)),
            pl.BlockSpec((1, O1), lambda bb, t: (0, 0)),
        ],
        out_specs=[
            pl.BlockSpec((RT, O1), lambda bb, t: (bb * (N2 // RT) + t, 0)),
            pl.BlockSpec((1, O1), lambda bb, t: (0, 0)),
            pl.BlockSpec((1, O1), lambda bb, t: (0, 0)),
        ],
        out_shape=[
            jax.ShapeDtypeStruct((NQ, O1), jnp.float32),
            jax.ShapeDtypeStruct((1, O1), jnp.float32),
            jax.ShapeDtypeStruct((1, O1), jnp.float32),
        ],
    )(y1a, features2, w1bT, b1r)

    mean1 = s1 / NQ
    var1 = q1 / NQ - mean1 * mean1
    a1 = gamma1.reshape(1, O1) / jnp.sqrt(var1 + EPS)
    c1 = beta1.reshape(1, O1) - mean1 * a1

    y2raw, s2, q2 = pl.pallas_call(
        _mlp2_body,
        grid=(NQ // RT,),
        in_specs=[
            pl.BlockSpec((RT, O1), lambda i: (i, 0)),
            pl.BlockSpec((1, O1), lambda i: (0, 0)),
            pl.BlockSpec((1, O1), lambda i: (0, 0)),
            pl.BlockSpec((O1, O2), lambda i: (0, 0)),
            pl.BlockSpec((1, O2), lambda i: (0, 0)),
        ],
        out_specs=[
            pl.BlockSpec((RT, O2), lambda i: (i, 0)),
            pl.BlockSpec((1, O2), lambda i: (0, 0)),
            pl.BlockSpec((1, O2), lambda i: (0, 0)),
        ],
        out_shape=[
            jax.ShapeDtypeStruct((NQ, O2), jnp.float32),
            jax.ShapeDtypeStruct((1, O2), jnp.float32),
            jax.ShapeDtypeStruct((1, O2), jnp.float32),
        ],
    )(y1raw, a1, c1, w2T, b2.reshape(1, O2))

    mean2 = s2 / NQ
    var2 = q2 / NQ - mean2 * mean2
    a2 = gamma2.reshape(1, O2) / jnp.sqrt(var2 + EPS)
    c2 = beta2.reshape(1, O2) - mean2 * a2

    out = pl.pallas_call(
        _fin_body,
        grid=(B, N2 // NT2),
        in_specs=[
            pl.BlockSpec((NT2, O2), lambda bb, t: (bb * (N2 // NT2) + t, 0)),
            pl.BlockSpec((1, O2), lambda bb, t: (0, 0)),
            pl.BlockSpec((1, O2), lambda bb, t: (0, 0)),
        ],
        out_specs=pl.BlockSpec((1, O2, NT2), lambda bb, t: (bb, 0, t)),
        out_shape=jax.ShapeDtypeStruct((B, O2, N2), jnp.float32),
    )(y2raw, a2, c2)
    return out


# speedup vs baseline: 12.7997x; 1.1196x over previous
import functools

import jax
import jax.numpy as jnp
from jax import lax
from jax.experimental import pallas as pl
from jax.experimental.pallas import tpu as pltpu
from jax.experimental.pallas import tpu_sc as plsc

B, N1, N2 = 16, 1024, 4096
C1, C2 = 512, 256
O1, O2 = 512, 256
NQ = B * N2
EPS = 1e-3

QT = 256
RT = 512
NT2 = 256
NW = 32
QPW = NQ // NW
QCH = 16
NCH = QPW // QCH


def _gtable_body(f1_ref, w_ref, g_ref):
    g_ref[...] = lax.dot_general(
        f1_ref[0].astype(jnp.bfloat16), w_ref[...].astype(jnp.bfloat16),
        (((0,), (0,)), ((), ())), preferred_element_type=jnp.float32)


def _nn3_body(p2_ref, p1_ref, i0_ref, i1_ref, i2_ref, wexp_ref):
    b = pl.program_id(0)
    q = p2_ref[0]
    kpts = p1_ref[0]
    d2 = jnp.zeros((QT, N1), jnp.float32)
    for j in range(3):
        diff = q[:, j:j + 1] - kpts[j:j + 1, :]
        d2 = d2 + diff * diff
    lane = lax.broadcasted_iota(jnp.int32, (QT, N1), 1)
    cur = d2
    mins, args = [], []
    for _ in range(3):
        m = jnp.min(cur, axis=1, keepdims=True)
        am = jnp.min(jnp.where(cur == m, lane, jnp.int32(2 ** 30)),
                     axis=1, keepdims=True)
        cur = jnp.where(lane == am, jnp.float32(jnp.inf), cur)
        mins.append(m)
        args.append(am)
    invs = []
    for m in mins:
        d = jnp.sqrt(jnp.maximum(m, 0.0))
        dd = d * d
        dd = jnp.where(dd < 1e-10, jnp.float32(1e-10), dd)
        invs.append(1.0 / dd)
    norm = (invs[0] + invs[1]) + invs[2]
    base = b * N1
    i0_ref[0] = args[0] + base
    i1_ref[0] = args[1] + base
    i2_ref[0] = args[2] + base
    wexp_ref[0] = jnp.concatenate(
        [jnp.broadcast_to(iv / norm, (QT, 16)) for iv in invs], axis=1)


def _interp_body(g_hbm, i0_hbm, i1_hbm, i2_hbm, wexp_hbm, out_hbm,
                 i0v, i1v, i2v,
                 ra0, ra1, ra2, wva, ova,
                 rb0, rb1, rb2, wvb, ovb,
                 gsa, gsb, osa, osb):
    cid = lax.axis_index("c")
    sid = lax.axis_index("s")
    wid = sid * 2 + cid
    base = wid * QPW
    pltpu.sync_copy(i0_hbm.at[pl.ds(base, QPW)], i0v)
    pltpu.sync_copy(i1_hbm.at[pl.ds(base, QPW)], i1v)
    pltpu.sync_copy(i2_hbm.at[pl.ds(base, QPW)], i2v)

    def fire(ci, rr0, rr1, rr2, wv, gs):
        cb = ci * QCH
        pltpu.async_copy(g_hbm.at[i0v.at[pl.ds(cb, QCH)]], rr0, gs)
        pltpu.async_copy(g_hbm.at[i1v.at[pl.ds(cb, QCH)]], rr1, gs)
        pltpu.async_copy(g_hbm.at[i2v.at[pl.ds(cb, QCH)]], rr2, gs)
        pltpu.async_copy(wexp_hbm.at[pl.ds((base + cb) * 48, QCH * 48)],
                         wv, gs)

    def drain_gather(rr0, rr1, rr2, wv, gs):
        pltpu.make_async_copy(g_hbm.at[pl.ds(0, QCH)], rr0, gs).wait()
        pltpu.make_async_copy(g_hbm.at[pl.ds(0, QCH)], rr1, gs).wait()
        pltpu.make_async_copy(g_hbm.at[pl.ds(0, QCH)], rr2, gs).wait()
        pltpu.make_async_copy(wexp_hbm.at[pl.ds(0, QCH * 48)], wv, gs).wait()

    def process(ci, rr0, rr1, rr2, wv, ov, gs, os,
                nr0, nr1, nr2, nwv, ngs):
        @pl.when(ci + 1 < NCH)
        def _():
            fire(ci + 1, nr0, nr1, nr2, nwv, ngs)

        drain_gather(rr0, rr1, rr2, wv, gs)

        @pl.when(ci >= 2)
        def _():
            pltpu.make_async_copy(ov, out_hbm.at[pl.ds(0, QCH)], os).wait()

        def qloop(qi, carry2):
            s0 = wv[pl.ds(qi * 48, 16)]
            s1 = wv[pl.ds(qi * 48 + 16, 16)]
            s2 = wv[pl.ds(qi * 48 + 32, 16)]
            for ch in range(O1 // 16):
                sl = pl.ds(ch * 16, 16)
                ov[qi, sl] = (s0 * rr0[qi, sl] + s1 * rr1[qi, sl]) \
                    + s2 * rr2[qi, sl]
            return carry2

        lax.fori_loop(0, QCH, qloop, 0)
        pltpu.async_copy(ov, out_hbm.at[pl.ds(base + ci * QCH, QCH)], os)

    fire(0, ra0, ra1, ra2, wva, gsa)

    def chunk(ci, carry):
        even = lax.rem(ci, 2) == 0

        @pl.when(even)
        def _():
            process(ci, ra0, ra1, ra2, wva, ova, gsa, osa,
                    rb0, rb1, rb2, wvb, gsb)

        @pl.when(jnp.logical_not(even))
        def _():
            process(ci, rb0, rb1, rb2, wvb, ovb, gsb, osb,
                    ra0, ra1, ra2, wva, gsa)

        return carry

    lax.fori_loop(0, NCH, chunk, 0)
    pltpu.make_async_copy(ova, out_hbm.at[pl.ds(0, QCH)], osa).wait()
    pltpu.make_async_copy(ovb, out_hbm.at[pl.ds(0, QCH)], osb).wait()


def _mlp1_body(y1a_ref, f2_ref, w1b_ref, b1_ref, y_ref, s_ref, q_ref):
    y = y1a_ref[...] + lax.dot_general(
        f2_ref[0].astype(jnp.bfloat16), w1b_ref[...].astype(jnp.bfloat16),
        (((0,), (0,)), ((), ())), preferred_element_type=jnp.float32)
    y = y + b1_ref[...]
    y_ref[...] = y

    @pl.when((pl.program_id(0) == 0) & (pl.program_id(1) == 0))
    def _():
        s_ref[...] = jnp.zeros_like(s_ref)
        q_ref[...] = jnp.zeros_like(q_ref)

    s_ref[...] += jnp.sum(y, axis=0, keepdims=True)
    q_ref[...] += jnp.sum(y * y, axis=0, keepdims=True)


def _mlp2_body(y1_ref, a1_ref, c1_ref, w2_ref, b2_ref, y2_ref, s_ref, q_ref):
    h = jnp.maximum(y1_ref[...] * a1_ref[...] + c1_ref[...], 0.0)
    y = jnp.dot(h.astype(jnp.bfloat16), w2_ref[...].astype(jnp.bfloat16),
                preferred_element_type=jnp.float32)
    y = y + b2_ref[...]
    y2_ref[...] = y

    @pl.when(pl.program_id(0) == 0)
    def _():
        s_ref[...] = jnp.zeros_like(s_ref)
        q_ref[...] = jnp.zeros_like(q_ref)

    s_ref[...] += jnp.sum(y, axis=0, keepdims=True)
    q_ref[...] += jnp.sum(y * y, axis=0, keepdims=True)


def _fin_body(y2_ref, a2_ref, c2_ref, o_ref):
    h = jnp.maximum(y2_ref[...] * a2_ref[...] + c2_ref[...], 0.0)
    o_ref[0] = h.T


def kernel(points1, points2, features1, features2, W1, b1, gamma1, beta1,
           W2, b2, gamma2, beta2):
    p2t = jnp.transpose(points2, (0, 2, 1))
    w1aT = jnp.transpose(W1[:, :C1])
    w1bT = jnp.transpose(W1[:, C1:])
    w2T = jnp.transpose(W2)
    b1r = b1.reshape(1, O1)

    g = pl.pallas_call(
        _gtable_body,
        grid=(B,),
        in_specs=[
            pl.BlockSpec((1, C1, N1), lambda bb: (bb, 0, 0)),
            pl.BlockSpec((C1, O1), lambda bb: (0, 0)),
        ],
        out_specs=pl.BlockSpec((N1, O1), lambda bb: (bb, 0)),
        out_shape=jax.ShapeDtypeStruct((B * N1, O1), jnp.float32),
    )(features1, w1aT)

    nn = pl.pallas_call(
        _nn3_body,
        grid=(B, N2 // QT),
        in_specs=[
            pl.BlockSpec((1, QT, 3), lambda bb, t: (bb, t, 0)),
            pl.BlockSpec((1, 3, N1), lambda bb, t: (bb, 0, 0)),
        ],
        out_specs=(
            [pl.BlockSpec((1, QT, 1), lambda bb, t: (bb, t, 0))] * 3
            + [pl.BlockSpec((1, QT, 48), lambda bb, t: (bb, t, 0))]),
        out_shape=(
            [jax.ShapeDtypeStruct((B, N2, 1), jnp.int32)] * 3
            + [jax.ShapeDtypeStruct((B, N2, 48), jnp.float32)]),
    )(p2t, points1)
    i0, i1, i2 = (x.reshape(NQ) for x in nn[:3])
    wexp = nn[3].reshape(NQ * 48)

    mesh = plsc.VectorSubcoreMesh(core_axis_name="c", subcore_axis_name="s")
    interp = functools.partial(
        pl.kernel,
        mesh=mesh,
        out_type=jax.ShapeDtypeStruct((NQ, O1), jnp.float32),
        scratch_types=(
            [pltpu.VMEM((QPW,), jnp.int32)] * 3
            + ([pltpu.VMEM((QCH, O1), jnp.float32)] * 3
               + [pltpu.VMEM((QCH * 48,), jnp.float32)]
               + [pltpu.VMEM((QCH, O1), jnp.float32)]) * 2
            + [pltpu.SemaphoreType.DMA] * 4
        ),
    )(_interp_body)
    y1a = interp(g, i0, i1, i2, wexp)

    y1raw, s1, q1 = pl.pallas_call(
        _mlp1_body,
        grid=(B, N2 // RT),
        in_specs=[
            pl.BlockSpec((RT, O1), lambda bb, t: (bb * (N2 // RT) + t, 0)),
            pl.BlockSpec((1, C2, RT), lambda bb, t: (bb, 0, t)),
            pl.BlockSpec((C2, O1), lambda bb, t: (0, 0)),
            pl.BlockSpec((1, O1), lambda bb, t: (0, 0)),
        ],
        out_specs=[
            pl.BlockSpec((RT, O1), lambda bb, t: (bb * (N2 // RT) + t, 0)),
            pl.BlockSpec((1, O1), lambda bb, t: (0, 0)),
            pl.BlockSpec((1, O1), lambda bb, t: (0, 0)),
        ],
        out_shape=[
            jax.ShapeDtypeStruct((NQ, O1), jnp.float32),
            jax.ShapeDtypeStruct((1, O1), jnp.float32),
            jax.ShapeDtypeStruct((1, O1), jnp.float32),
        ],
    )(y1a, features2, w1bT, b1r)

    mean1 = s1 / NQ
    var1 = q1 / NQ - mean1 * mean1
    a1 = gamma1.reshape(1, O1) / jnp.sqrt(var1 + EPS)
    c1 = beta1.reshape(1, O1) - mean1 * a1

    y2raw, s2, q2 = pl.pallas_call(
        _mlp2_body,
        grid=(NQ // RT,),
        in_specs=[
            pl.BlockSpec((RT, O1), lambda i: (i, 0)),
            pl.BlockSpec((1, O1), lambda i: (0, 0)),
            pl.BlockSpec((1, O1), lambda i: (0, 0)),
            pl.BlockSpec((O1, O2), lambda i: (0, 0)),
            pl.BlockSpec((1, O2), lambda i: (0, 0)),
        ],
        out_specs=[
            pl.BlockSpec((RT, O2), lambda i: (i, 0)),
            pl.BlockSpec((1, O2), lambda i: (0, 0)),
            pl.BlockSpec((1, O2), lambda i: (0, 0)),
        ],
        out_shape=[
            jax.ShapeDtypeStruct((NQ, O2), jnp.float32),
            jax.ShapeDtypeStruct((1, O2), jnp.float32),
            jax.ShapeDtypeStruct((1, O2), jnp.float32),
        ],
    )(y1raw, a1, c1, w2T, b2.reshape(1, O2))

    mean2 = s2 / NQ
    var2 = q2 / NQ - mean2 * mean2
    a2 = gamma2.reshape(1, O2) / jnp.sqrt(var2 + EPS)
    c2 = beta2.reshape(1, O2) - mean2 * a2

    out = pl.pallas_call(
        _fin_body,
        grid=(B, N2 // NT2),
        in_specs=[
            pl.BlockSpec((NT2, O2), lambda bb, t: (bb * (N2 // NT2) + t, 0)),
            pl.BlockSpec((1, O2), lambda bb, t: (0, 0)),
            pl.BlockSpec((1, O2), lambda bb, t: (0, 0)),
        ],
        out_specs=pl.BlockSpec((1, O2, NT2), lambda bb, t: (bb, 0, t)),
        out_shape=jax.ShapeDtypeStruct((B, O2, N2), jnp.float32),
    )(y2raw, a2, c2)
    return out


# speedup vs baseline: 13.2428x; 1.0346x over previous
import functools

import jax
import jax.numpy as jnp
from jax import lax
from jax.experimental import pallas as pl
from jax.experimental.pallas import tpu as pltpu
from jax.experimental.pallas import tpu_sc as plsc

B, N1, N2 = 16, 1024, 4096
C1, C2 = 512, 256
O1, O2 = 512, 256
NQ = B * N2
EPS = 1e-3

QT = 256
RT = 512
NT2 = 256
NW = 32
QPW = NQ // NW
QCH = 16
NCH = QPW // QCH


def _gtable_body(f1_ref, w_ref, g_ref):
    g_ref[...] = lax.dot_general(
        f1_ref[0].astype(jnp.bfloat16), w_ref[...].astype(jnp.bfloat16),
        (((0,), (0,)), ((), ())), preferred_element_type=jnp.float32)


def _nn3_body(p2_ref, p1_ref, i0_ref, i1_ref, i2_ref, wexp_ref):
    b = pl.program_id(0)
    q = p2_ref[0]
    kpts = p1_ref[0]
    d2 = jnp.zeros((QT, N1), jnp.float32)
    for j in range(3):
        diff = q[:, j:j + 1] - kpts[j:j + 1, :]
        d2 = d2 + diff * diff
    lane = lax.broadcasted_iota(jnp.int32, (QT, N1), 1)
    cur = d2
    mins, args = [], []
    for _ in range(3):
        m = jnp.min(cur, axis=1, keepdims=True)
        am = jnp.min(jnp.where(cur == m, lane, jnp.int32(2 ** 30)),
                     axis=1, keepdims=True)
        cur = jnp.where(lane == am, jnp.float32(jnp.inf), cur)
        mins.append(m)
        args.append(am)
    invs = []
    for m in mins:
        d = jnp.sqrt(jnp.maximum(m, 0.0))
        dd = d * d
        dd = jnp.where(dd < 1e-10, jnp.float32(1e-10), dd)
        invs.append(1.0 / dd)
    norm = (invs[0] + invs[1]) + invs[2]
    base = b * N1
    i0_ref[0] = args[0] + base
    i1_ref[0] = args[1] + base
    i2_ref[0] = args[2] + base
    wexp_ref[0] = jnp.concatenate(
        [jnp.broadcast_to(iv / norm, (QT, 16)) for iv in invs], axis=1)


def _interp_body(g_hbm, i0_hbm, i1_hbm, i2_hbm, wexp_hbm, out_hbm,
                 i0v, i1v, i2v,
                 ra0, ra1, ra2, wva, ova,
                 rb0, rb1, rb2, wvb, ovb,
                 gsa, gsb, osa, osb):
    cid = lax.axis_index("c")
    sid = lax.axis_index("s")
    wid = sid * 2 + cid
    base = wid * QPW
    pltpu.sync_copy(i0_hbm.at[pl.ds(base, QPW)], i0v)
    pltpu.sync_copy(i1_hbm.at[pl.ds(base, QPW)], i1v)
    pltpu.sync_copy(i2_hbm.at[pl.ds(base, QPW)], i2v)

    def fire(ci, rr0, rr1, rr2, wv, gs):
        cb = ci * QCH
        pltpu.async_copy(g_hbm.at[i0v.at[pl.ds(cb, QCH)]], rr0, gs)
        pltpu.async_copy(g_hbm.at[i1v.at[pl.ds(cb, QCH)]], rr1, gs)
        pltpu.async_copy(g_hbm.at[i2v.at[pl.ds(cb, QCH)]], rr2, gs)
        pltpu.async_copy(wexp_hbm.at[pl.ds((base + cb) * 48, QCH * 48)],
                         wv, gs)

    def drain_gather(rr0, rr1, rr2, wv, gs):
        pltpu.make_async_copy(g_hbm.at[pl.ds(0, QCH)], rr0, gs).wait()
        pltpu.make_async_copy(g_hbm.at[pl.ds(0, QCH)], rr1, gs).wait()
        pltpu.make_async_copy(g_hbm.at[pl.ds(0, QCH)], rr2, gs).wait()
        pltpu.make_async_copy(wexp_hbm.at[pl.ds(0, QCH * 48)], wv, gs).wait()

    def process(ci, rr0, rr1, rr2, wv, ov, gs, os,
                nr0, nr1, nr2, nwv, ngs):
        @pl.when(ci + 1 < NCH)
        def _():
            fire(ci + 1, nr0, nr1, nr2, nwv, ngs)

        drain_gather(rr0, rr1, rr2, wv, gs)

        @pl.when(ci >= 2)
        def _():
            pltpu.make_async_copy(ov, out_hbm.at[pl.ds(0, QCH)], os).wait()

        def qloop(qi, carry2):
            s0 = wv[pl.ds(qi * 48, 16)]
            s1 = wv[pl.ds(qi * 48 + 16, 16)]
            s2 = wv[pl.ds(qi * 48 + 32, 16)]
            for ch in range(O1 // 16):
                sl = pl.ds(ch * 16, 16)
                ov[qi, sl] = (s0 * rr0[qi, sl] + s1 * rr1[qi, sl]) \
                    + s2 * rr2[qi, sl]
            return carry2

        lax.fori_loop(0, QCH, qloop, 0)
        pltpu.async_copy(ov, out_hbm.at[pl.ds(base + ci * QCH, QCH)], os)

    fire(0, ra0, ra1, ra2, wva, gsa)

    def chunk(ci, carry):
        even = lax.rem(ci, 2) == 0

        @pl.when(even)
        def _():
            process(ci, ra0, ra1, ra2, wva, ova, gsa, osa,
                    rb0, rb1, rb2, wvb, gsb)

        @pl.when(jnp.logical_not(even))
        def _():
            process(ci, rb0, rb1, rb2, wvb, ovb, gsb, osb,
                    ra0, ra1, ra2, wva, gsa)

        return carry

    lax.fori_loop(0, NCH, chunk, 0)
    pltpu.make_async_copy(ova, out_hbm.at[pl.ds(0, QCH)], osa).wait()
    pltpu.make_async_copy(ovb, out_hbm.at[pl.ds(0, QCH)], osb).wait()


def _mlp1_body(y1a_ref, f2_ref, w1b_ref, b1_ref, y_ref, s_ref, q_ref):
    y = y1a_ref[...] + lax.dot_general(
        f2_ref[0].astype(jnp.bfloat16), w1b_ref[...].astype(jnp.bfloat16),
        (((0,), (0,)), ((), ())), preferred_element_type=jnp.float32)
    y = y + b1_ref[...]
    y_ref[...] = y.astype(jnp.bfloat16)

    @pl.when((pl.program_id(0) == 0) & (pl.program_id(1) == 0))
    def _():
        s_ref[...] = jnp.zeros_like(s_ref)
        q_ref[...] = jnp.zeros_like(q_ref)

    s_ref[...] += jnp.sum(y, axis=0, keepdims=True)
    q_ref[...] += jnp.sum(y * y, axis=0, keepdims=True)


def _mlp2_body(y1_ref, a1_ref, c1_ref, w2_ref, b2_ref, y2_ref, s_ref, q_ref):
    y1 = y1_ref[...].astype(jnp.float32)
    h = jnp.maximum(y1 * a1_ref[...] + c1_ref[...], 0.0)
    y = jnp.dot(h.astype(jnp.bfloat16), w2_ref[...].astype(jnp.bfloat16),
                preferred_element_type=jnp.float32)
    y = y + b2_ref[...]
    y2_ref[...] = y.astype(jnp.bfloat16)

    @pl.when(pl.program_id(0) == 0)
    def _():
        s_ref[...] = jnp.zeros_like(s_ref)
        q_ref[...] = jnp.zeros_like(q_ref)

    s_ref[...] += jnp.sum(y, axis=0, keepdims=True)
    q_ref[...] += jnp.sum(y * y, axis=0, keepdims=True)


def _fin_body(y2_ref, a2_ref, c2_ref, o_ref):
    y2 = y2_ref[...].astype(jnp.float32)
    h = jnp.maximum(y2 * a2_ref[...] + c2_ref[...], 0.0)
    o_ref[0] = h.T


def kernel(points1, points2, features1, features2, W1, b1, gamma1, beta1,
           W2, b2, gamma2, beta2):
    p2t = jnp.transpose(points2, (0, 2, 1))
    w1aT = jnp.transpose(W1[:, :C1])
    w1bT = jnp.transpose(W1[:, C1:])
    w2T = jnp.transpose(W2)
    b1r = b1.reshape(1, O1)

    g = pl.pallas_call(
        _gtable_body,
        grid=(B,),
        in_specs=[
            pl.BlockSpec((1, C1, N1), lambda bb: (bb, 0, 0)),
            pl.BlockSpec((C1, O1), lambda bb: (0, 0)),
        ],
        out_specs=pl.BlockSpec((N1, O1), lambda bb: (bb, 0)),
        out_shape=jax.ShapeDtypeStruct((B * N1, O1), jnp.float32),
    )(features1, w1aT)

    nn = pl.pallas_call(
        _nn3_body,
        grid=(B, N2 // QT),
        in_specs=[
            pl.BlockSpec((1, QT, 3), lambda bb, t: (bb, t, 0)),
            pl.BlockSpec((1, 3, N1), lambda bb, t: (bb, 0, 0)),
        ],
        out_specs=(
            [pl.BlockSpec((1, QT, 1), lambda bb, t: (bb, t, 0))] * 3
            + [pl.BlockSpec((1, QT, 48), lambda bb, t: (bb, t, 0))]),
        out_shape=(
            [jax.ShapeDtypeStruct((B, N2, 1), jnp.int32)] * 3
            + [jax.ShapeDtypeStruct((B, N2, 48), jnp.float32)]),
    )(p2t, points1)
    i0, i1, i2 = (x.reshape(NQ) for x in nn[:3])
    wexp = nn[3].reshape(NQ * 48)

    mesh = plsc.VectorSubcoreMesh(core_axis_name="c", subcore_axis_name="s")
    interp = functools.partial(
        pl.kernel,
        mesh=mesh,
        out_type=jax.ShapeDtypeStruct((NQ, O1), jnp.float32),
        scratch_types=(
            [pltpu.VMEM((QPW,), jnp.int32)] * 3
            + ([pltpu.VMEM((QCH, O1), jnp.float32)] * 3
               + [pltpu.VMEM((QCH * 48,), jnp.float32)]
               + [pltpu.VMEM((QCH, O1), jnp.float32)]) * 2
            + [pltpu.SemaphoreType.DMA] * 4
        ),
    )(_interp_body)
    y1a = interp(g, i0, i1, i2, wexp)

    y1raw, s1, q1 = pl.pallas_call(
        _mlp1_body,
        grid=(B, N2 // RT),
        in_specs=[
            pl.BlockSpec((RT, O1), lambda bb, t: (bb * (N2 // RT) + t, 0)),
            pl.BlockSpec((1, C2, RT), lambda bb, t: (bb, 0, t)),
            pl.BlockSpec((C2, O1), lambda bb, t: (0, 0)),
            pl.BlockSpec((1, O1), lambda bb, t: (0, 0)),
        ],
        out_specs=[
            pl.BlockSpec((RT, O1), lambda bb, t: (bb * (N2 // RT) + t, 0)),
            pl.BlockSpec((1, O1), lambda bb, t: (0, 0)),
            pl.BlockSpec((1, O1), lambda bb, t: (0, 0)),
        ],
        out_shape=[
            jax.ShapeDtypeStruct((NQ, O1), jnp.bfloat16),
            jax.ShapeDtypeStruct((1, O1), jnp.float32),
            jax.ShapeDtypeStruct((1, O1), jnp.float32),
        ],
    )(y1a, features2, w1bT, b1r)

    mean1 = s1 / NQ
    var1 = q1 / NQ - mean1 * mean1
    a1 = gamma1.reshape(1, O1) / jnp.sqrt(var1 + EPS)
    c1 = beta1.reshape(1, O1) - mean1 * a1

    y2raw, s2, q2 = pl.pallas_call(
        _mlp2_body,
        grid=(NQ // RT,),
        in_specs=[
            pl.BlockSpec((RT, O1), lambda i: (i, 0)),
            pl.BlockSpec((1, O1), lambda i: (0, 0)),
            pl.BlockSpec((1, O1), lambda i: (0, 0)),
            pl.BlockSpec((O1, O2), lambda i: (0, 0)),
            pl.BlockSpec((1, O2), lambda i: (0, 0)),
        ],
        out_specs=[
            pl.BlockSpec((RT, O2), lambda i: (i, 0)),
            pl.BlockSpec((1, O2), lambda i: (0, 0)),
            pl.BlockSpec((1, O2), lambda i: (0, 0)),
        ],
        out_shape=[
            jax.ShapeDtypeStruct((NQ, O2), jnp.bfloat16),
            jax.ShapeDtypeStruct((1, O2), jnp.float32),
            jax.ShapeDtypeStruct((1, O2), jnp.float32),
        ],
    )(y1raw, a1, c1, w2T, b2.reshape(1, O2))

    mean2 = s2 / NQ
    var2 = q2 / NQ - mean2 * mean2
    a2 = gamma2.reshape(1, O2) / jnp.sqrt(var2 + EPS)
    c2 = beta2.reshape(1, O2) - mean2 * a2

    out = pl.pallas_call(
        _fin_body,
        grid=(B, N2 // NT2),
        in_specs=[
            pl.BlockSpec((NT2, O2), lambda bb, t: (bb * (N2 // NT2) + t, 0)),
            pl.BlockSpec((1, O2), lambda bb, t: (0, 0)),
            pl.BlockSpec((1, O2), lambda bb, t: (0, 0)),
        ],
        out_specs=pl.BlockSpec((1, O2, NT2), lambda bb, t: (bb, 0, t)),
        out_shape=jax.ShapeDtypeStruct((B, O2, N2), jnp.float32),
    )(y2raw, a2, c2)
    return out


# speedup vs baseline: 14.1521x; 1.0687x over previous
import functools

import jax
import jax.numpy as jnp
from jax import lax
from jax.experimental import pallas as pl
from jax.experimental.pallas import tpu as pltpu
from jax.experimental.pallas import tpu_sc as plsc

B, N1, N2 = 16, 1024, 4096
C1, C2 = 512, 256
O1, O2 = 512, 256
NQ = B * N2
EPS = 1e-3

QT = 256
RT = 512
NT2 = 256
NW = 32
QPW = NQ // NW
QCH = 16
NCH = QPW // QCH


def _gtable_body(f1_ref, w_ref, g_ref):
    g_ref[...] = lax.dot_general(
        f1_ref[0].astype(jnp.bfloat16), w_ref[...].astype(jnp.bfloat16),
        (((0,), (0,)), ((), ())), preferred_element_type=jnp.float32)


def _nn3_body(p2_ref, p1_ref, i0_ref, i1_ref, i2_ref, wexp_ref):
    b = pl.program_id(0)
    q = p2_ref[0]
    kpts = p1_ref[0]
    d2 = jnp.zeros((QT, N1), jnp.float32)
    for j in range(3):
        diff = q[:, j:j + 1] - kpts[j:j + 1, :]
        d2 = d2 + diff * diff
    lanef = lax.broadcasted_iota(jnp.int32, (QT, N1), 1).astype(jnp.float32)
    cur = d2
    mins, args = [], []
    for t in range(3):
        m = jnp.min(cur, axis=1, keepdims=True)
        amf = jnp.min(jnp.where(cur == m, lanef, jnp.float32(2.0 ** 30)),
                      axis=1, keepdims=True)
        if t < 2:
            cur = jnp.where(lanef == amf, jnp.float32(jnp.inf), cur)
        mins.append(m)
        args.append(amf.astype(jnp.int32))
    invs = []
    for m in mins:
        d = jnp.sqrt(jnp.maximum(m, 0.0))
        dd = d * d
        dd = jnp.where(dd < 1e-10, jnp.float32(1e-10), dd)
        invs.append(1.0 / dd)
    norm = (invs[0] + invs[1]) + invs[2]
    base = b * N1
    i0_ref[0] = args[0] + base
    i1_ref[0] = args[1] + base
    i2_ref[0] = args[2] + base
    wexp_ref[0] = jnp.concatenate(
        [jnp.broadcast_to(iv / norm, (QT, 16)) for iv in invs], axis=1)


def _interp_body(g_hbm, i0_hbm, i1_hbm, i2_hbm, wexp_hbm, out_hbm,
                 i0v, i1v, i2v,
                 ra0, ra1, ra2, wva, ova,
                 rb0, rb1, rb2, wvb, ovb,
                 gsa, gsb, osa, osb):
    cid = lax.axis_index("c")
    sid = lax.axis_index("s")
    wid = sid * 2 + cid
    base = wid * QPW
    pltpu.sync_copy(i0_hbm.at[pl.ds(base, QPW)], i0v)
    pltpu.sync_copy(i1_hbm.at[pl.ds(base, QPW)], i1v)
    pltpu.sync_copy(i2_hbm.at[pl.ds(base, QPW)], i2v)

    def fire(ci, rr0, rr1, rr2, wv, gs):
        cb = ci * QCH
        pltpu.async_copy(g_hbm.at[i0v.at[pl.ds(cb, QCH)]], rr0, gs)
        pltpu.async_copy(g_hbm.at[i1v.at[pl.ds(cb, QCH)]], rr1, gs)
        pltpu.async_copy(g_hbm.at[i2v.at[pl.ds(cb, QCH)]], rr2, gs)
        pltpu.async_copy(wexp_hbm.at[pl.ds((base + cb) * 48, QCH * 48)],
                         wv, gs)

    def drain_gather(rr0, rr1, rr2, wv, gs):
        pltpu.make_async_copy(g_hbm.at[pl.ds(0, QCH)], rr0, gs).wait()
        pltpu.make_async_copy(g_hbm.at[pl.ds(0, QCH)], rr1, gs).wait()
        pltpu.make_async_copy(g_hbm.at[pl.ds(0, QCH)], rr2, gs).wait()
        pltpu.make_async_copy(wexp_hbm.at[pl.ds(0, QCH * 48)], wv, gs).wait()

    def process(ci, rr0, rr1, rr2, wv, ov, gs, os,
                nr0, nr1, nr2, nwv, ngs):
        @pl.when(ci + 1 < NCH)
        def _():
            fire(ci + 1, nr0, nr1, nr2, nwv, ngs)

        drain_gather(rr0, rr1, rr2, wv, gs)

        @pl.when(ci >= 2)
        def _():
            pltpu.make_async_copy(ov, out_hbm.at[pl.ds(0, QCH)], os).wait()

        def qloop(qi, carry2):
            s0 = wv[pl.ds(qi * 48, 16)]
            s1 = wv[pl.ds(qi * 48 + 16, 16)]
            s2 = wv[pl.ds(qi * 48 + 32, 16)]
            for ch in range(O1 // 16):
                sl = pl.ds(ch * 16, 16)
                ov[qi, sl] = (s0 * rr0[qi, sl] + s1 * rr1[qi, sl]) \
                    + s2 * rr2[qi, sl]
            return carry2

        lax.fori_loop(0, QCH, qloop, 0)
        pltpu.async_copy(ov, out_hbm.at[pl.ds(base + ci * QCH, QCH)], os)

    fire(0, ra0, ra1, ra2, wva, gsa)

    def chunk(ci, carry):
        even = lax.rem(ci, 2) == 0

        @pl.when(even)
        def _():
            process(ci, ra0, ra1, ra2, wva, ova, gsa, osa,
                    rb0, rb1, rb2, wvb, gsb)

        @pl.when(jnp.logical_not(even))
        def _():
            process(ci, rb0, rb1, rb2, wvb, ovb, gsb, osb,
                    ra0, ra1, ra2, wva, gsa)

        return carry

    lax.fori_loop(0, NCH, chunk, 0)
    pltpu.make_async_copy(ova, out_hbm.at[pl.ds(0, QCH)], osa).wait()
    pltpu.make_async_copy(ovb, out_hbm.at[pl.ds(0, QCH)], osb).wait()


def _mlp1_body(y1a_ref, f2_ref, w1b_ref, b1_ref, y_ref, s_ref, q_ref):
    y = y1a_ref[...] + lax.dot_general(
        f2_ref[0].astype(jnp.bfloat16), w1b_ref[...].astype(jnp.bfloat16),
        (((0,), (0,)), ((), ())), preferred_element_type=jnp.float32)
    y = y + b1_ref[...]
    y_ref[...] = y.astype(jnp.bfloat16)

    @pl.when((pl.program_id(0) == 0) & (pl.program_id(1) == 0))
    def _():
        s_ref[...] = jnp.zeros_like(s_ref)
        q_ref[...] = jnp.zeros_like(q_ref)

    s_ref[...] += jnp.sum(y, axis=0, keepdims=True)
    q_ref[...] += jnp.sum(y * y, axis=0, keepdims=True)


def _mlp2_body(y1_ref, a1_ref, c1_ref, w2_ref, b2_ref, y2_ref, s_ref, q_ref):
    y1 = y1_ref[...].astype(jnp.float32)
    h = jnp.maximum(y1 * a1_ref[...] + c1_ref[...], 0.0)
    y = jnp.dot(h.astype(jnp.bfloat16), w2_ref[...].astype(jnp.bfloat16),
                preferred_element_type=jnp.float32)
    y = y + b2_ref[...]
    y2_ref[...] = y.astype(jnp.bfloat16)

    @pl.when(pl.program_id(0) == 0)
    def _():
        s_ref[...] = jnp.zeros_like(s_ref)
        q_ref[...] = jnp.zeros_like(q_ref)

    s_ref[...] += jnp.sum(y, axis=0, keepdims=True)
    q_ref[...] += jnp.sum(y * y, axis=0, keepdims=True)


def _fin_body(y2_ref, a2_ref, c2_ref, o_ref):
    y2 = y2_ref[...].astype(jnp.float32)
    h = jnp.maximum(y2 * a2_ref[...] + c2_ref[...], 0.0)
    o_ref[0] = h.T


def kernel(points1, points2, features1, features2, W1, b1, gamma1, beta1,
           W2, b2, gamma2, beta2):
    p2t = jnp.transpose(points2, (0, 2, 1))
    w1aT = jnp.transpose(W1[:, :C1])
    w1bT = jnp.transpose(W1[:, C1:])
    w2T = jnp.transpose(W2)
    b1r = b1.reshape(1, O1)

    g = pl.pallas_call(
        _gtable_body,
        grid=(B,),
        in_specs=[
            pl.BlockSpec((1, C1, N1), lambda bb: (bb, 0, 0)),
            pl.BlockSpec((C1, O1), lambda bb: (0, 0)),
        ],
        out_specs=pl.BlockSpec((N1, O1), lambda bb: (bb, 0)),
        out_shape=jax.ShapeDtypeStruct((B * N1, O1), jnp.float32),
    )(features1, w1aT)

    nn = pl.pallas_call(
        _nn3_body,
        grid=(B, N2 // QT),
        in_specs=[
            pl.BlockSpec((1, QT, 3), lambda bb, t: (bb, t, 0)),
            pl.BlockSpec((1, 3, N1), lambda bb, t: (bb, 0, 0)),
        ],
        out_specs=(
            [pl.BlockSpec((1, QT, 1), lambda bb, t: (bb, t, 0))] * 3
            + [pl.BlockSpec((1, QT, 48), lambda bb, t: (bb, t, 0))]),
        out_shape=(
            [jax.ShapeDtypeStruct((B, N2, 1), jnp.int32)] * 3
            + [jax.ShapeDtypeStruct((B, N2, 48), jnp.float32)]),
    )(p2t, points1)
    i0, i1, i2 = (x.reshape(NQ) for x in nn[:3])
    wexp = nn[3].reshape(NQ * 48)

    mesh = plsc.VectorSubcoreMesh(core_axis_name="c", subcore_axis_name="s")
    interp = functools.partial(
        pl.kernel,
        mesh=mesh,
        out_type=jax.ShapeDtypeStruct((NQ, O1), jnp.float32),
        scratch_types=(
            [pltpu.VMEM((QPW,), jnp.int32)] * 3
            + ([pltpu.VMEM((QCH, O1), jnp.float32)] * 3
               + [pltpu.VMEM((QCH * 48,), jnp.float32)]
               + [pltpu.VMEM((QCH, O1), jnp.float32)]) * 2
            + [pltpu.SemaphoreType.DMA] * 4
        ),
    )(_interp_body)
    y1a = interp(g, i0, i1, i2, wexp)

    y1raw, s1, q1 = pl.pallas_call(
        _mlp1_body,
        grid=(B, N2 // RT),
        in_specs=[
            pl.BlockSpec((RT, O1), lambda bb, t: (bb * (N2 // RT) + t, 0)),
            pl.BlockSpec((1, C2, RT), lambda bb, t: (bb, 0, t)),
            pl.BlockSpec((C2, O1), lambda bb, t: (0, 0)),
            pl.BlockSpec((1, O1), lambda bb, t: (0, 0)),
        ],
        out_specs=[
            pl.BlockSpec((RT, O1), lambda bb, t: (bb * (N2 // RT) + t, 0)),
            pl.BlockSpec((1, O1), lambda bb, t: (0, 0)),
            pl.BlockSpec((1, O1), lambda bb, t: (0, 0)),
        ],
        out_shape=[
            jax.ShapeDtypeStruct((NQ, O1), jnp.bfloat16),
            jax.ShapeDtypeStruct((1, O1), jnp.float32),
            jax.ShapeDtypeStruct((1, O1), jnp.float32),
        ],
    )(y1a, features2, w1bT, b1r)

    mean1 = s1 / NQ
    var1 = q1 / NQ - mean1 * mean1
    a1 = gamma1.reshape(1, O1) / jnp.sqrt(var1 + EPS)
    c1 = beta1.reshape(1, O1) - mean1 * a1

    y2raw, s2, q2 = pl.pallas_call(
        _mlp2_body,
        grid=(NQ // RT,),
        in_specs=[
            pl.BlockSpec((RT, O1), lambda i: (i, 0)),
            pl.BlockSpec((1, O1), lambda i: (0, 0)),
            pl.BlockSpec((1, O1), lambda i: (0, 0)),
            pl.BlockSpec((O1, O2), lambda i: (0, 0)),
            pl.BlockSpec((1, O2), lambda i: (0, 0)),
        ],
        out_specs=[
            pl.BlockSpec((RT, O2), lambda i: (i, 0)),
            pl.BlockSpec((1, O2), lambda i: (0, 0)),
            pl.BlockSpec((1, O2), lambda i: (0, 0)),
        ],
        out_shape=[
            jax.ShapeDtypeStruct((NQ, O2), jnp.bfloat16),
            jax.ShapeDtypeStruct((1, O2), jnp.float32),
            jax.ShapeDtypeStruct((1, O2), jnp.float32),
        ],
    )(y1raw, a1, c1, w2T, b2.reshape(1, O2))

    mean2 = s2 / NQ
    var2 = q2 / NQ - mean2 * mean2
    a2 = gamma2.reshape(1, O2) / jnp.sqrt(var2 + EPS)
    c2 = beta2.reshape(1, O2) - mean2 * a2

    out = pl.pallas_call(
        _fin_body,
        grid=(B, N2 // NT2),
        in_specs=[
            pl.BlockSpec((NT2, O2), lambda bb, t: (bb * (N2 // NT2) + t, 0)),
            pl.BlockSpec((1, O2), lambda bb, t: (0, 0)),
            pl.BlockSpec((1, O2), lambda bb, t: (0, 0)),
        ],
        out_specs=pl.BlockSpec((1, O2, NT2), lambda bb, t: (bb, 0, t)),
        out_shape=jax.ShapeDtypeStruct((B, O2, N2), jnp.float32),
    )(y2raw, a2, c2)
    return out


# speedup vs baseline: 14.4236x; 1.0192x over previous
import functools

import jax
import jax.numpy as jnp
from jax import lax
from jax.experimental import pallas as pl
from jax.experimental.pallas import tpu as pltpu
from jax.experimental.pallas import tpu_sc as plsc

B, N1, N2 = 16, 1024, 4096
C1, C2 = 512, 256
O1, O2 = 512, 256
NQ = B * N2
EPS = 1e-3

QT = 512
RT = 512
NT2 = 256
NW = 32
QPW = NQ // NW
QCH = 16
NCH = QPW // QCH


def _gtable_body(f1_ref, w_ref, g_ref):
    g_ref[...] = lax.dot_general(
        f1_ref[0].astype(jnp.bfloat16), w_ref[...].astype(jnp.bfloat16),
        (((0,), (0,)), ((), ())), preferred_element_type=jnp.float32)


def _nn3_body(p2_ref, p1_ref, i0_ref, i1_ref, i2_ref, wexp_ref):
    b = pl.program_id(0)
    q = p2_ref[0]
    kpts = p1_ref[0]
    d2 = jnp.zeros((QT, N1), jnp.float32)
    for j in range(3):
        diff = q[:, j:j + 1] - kpts[j:j + 1, :]
        d2 = d2 + diff * diff
    lanef = lax.broadcasted_iota(jnp.int32, (QT, N1), 1).astype(jnp.float32)
    cur = d2
    mins, args = [], []
    for t in range(3):
        m = jnp.min(cur, axis=1, keepdims=True)
        amf = jnp.min(jnp.where(cur == m, lanef, jnp.float32(2.0 ** 30)),
                      axis=1, keepdims=True)
        if t < 2:
            cur = jnp.where(lanef == amf, jnp.float32(jnp.inf), cur)
        mins.append(m)
        args.append(amf.astype(jnp.int32))
    invs = []
    for m in mins:
        d = jnp.sqrt(jnp.maximum(m, 0.0))
        dd = d * d
        dd = jnp.where(dd < 1e-10, jnp.float32(1e-10), dd)
        invs.append(1.0 / dd)
    norm = (invs[0] + invs[1]) + invs[2]
    base = b * N1
    i0_ref[0] = args[0] + base
    i1_ref[0] = args[1] + base
    i2_ref[0] = args[2] + base
    wexp_ref[0] = jnp.concatenate(
        [jnp.broadcast_to(iv / norm, (QT, 16)) for iv in invs], axis=1)


def _interp_body(g_hbm, i0_hbm, i1_hbm, i2_hbm, wexp_hbm, out_hbm,
                 i0v, i1v, i2v,
                 ra0, ra1, ra2, wva, ova,
                 rb0, rb1, rb2, wvb, ovb,
                 gsa, gsb, osa, osb):
    cid = lax.axis_index("c")
    sid = lax.axis_index("s")
    wid = sid * 2 + cid
    base = wid * QPW
    pltpu.sync_copy(i0_hbm.at[pl.ds(base, QPW)], i0v)
    pltpu.sync_copy(i1_hbm.at[pl.ds(base, QPW)], i1v)
    pltpu.sync_copy(i2_hbm.at[pl.ds(base, QPW)], i2v)

    def fire(ci, rr0, rr1, rr2, wv, gs):
        cb = ci * QCH
        pltpu.async_copy(g_hbm.at[i0v.at[pl.ds(cb, QCH)]], rr0, gs)
        pltpu.async_copy(g_hbm.at[i1v.at[pl.ds(cb, QCH)]], rr1, gs)
        pltpu.async_copy(g_hbm.at[i2v.at[pl.ds(cb, QCH)]], rr2, gs)
        pltpu.async_copy(wexp_hbm.at[pl.ds((base + cb) * 48, QCH * 48)],
                         wv, gs)

    def drain_gather(rr0, rr1, rr2, wv, gs):
        pltpu.make_async_copy(g_hbm.at[pl.ds(0, QCH)], rr0, gs).wait()
        pltpu.make_async_copy(g_hbm.at[pl.ds(0, QCH)], rr1, gs).wait()
        pltpu.make_async_copy(g_hbm.at[pl.ds(0, QCH)], rr2, gs).wait()
        pltpu.make_async_copy(wexp_hbm.at[pl.ds(0, QCH * 48)], wv, gs).wait()

    def process(ci, rr0, rr1, rr2, wv, ov, gs, os,
                nr0, nr1, nr2, nwv, ngs):
        @pl.when(ci + 1 < NCH)
        def _():
            fire(ci + 1, nr0, nr1, nr2, nwv, ngs)

        drain_gather(rr0, rr1, rr2, wv, gs)

        @pl.when(ci >= 2)
        def _():
            pltpu.make_async_copy(ov, out_hbm.at[pl.ds(0, QCH)], os).wait()

        def qloop(qi, carry2):
            s0 = wv[pl.ds(qi * 48, 16)]
            s1 = wv[pl.ds(qi * 48 + 16, 16)]
            s2 = wv[pl.ds(qi * 48 + 32, 16)]
            for ch in range(O1 // 16):
                sl = pl.ds(ch * 16, 16)
                ov[qi, sl] = (s0 * rr0[qi, sl] + s1 * rr1[qi, sl]) \
                    + s2 * rr2[qi, sl]
            return carry2

        lax.fori_loop(0, QCH, qloop, 0)
        pltpu.async_copy(ov, out_hbm.at[pl.ds(base + ci * QCH, QCH)], os)

    fire(0, ra0, ra1, ra2, wva, gsa)

    def chunk(ci, carry):
        even = lax.rem(ci, 2) == 0

        @pl.when(even)
        def _():
            process(ci, ra0, ra1, ra2, wva, ova, gsa, osa,
                    rb0, rb1, rb2, wvb, gsb)

        @pl.when(jnp.logical_not(even))
        def _():
            process(ci, rb0, rb1, rb2, wvb, ovb, gsb, osb,
                    ra0, ra1, ra2, wva, gsa)

        return carry

    lax.fori_loop(0, NCH, chunk, 0)
    pltpu.make_async_copy(ova, out_hbm.at[pl.ds(0, QCH)], osa).wait()
    pltpu.make_async_copy(ovb, out_hbm.at[pl.ds(0, QCH)], osb).wait()


def _mlp1_body(y1a_ref, f2_ref, w1b_ref, b1_ref, y_ref, s_ref, q_ref):
    y = y1a_ref[...] + lax.dot_general(
        f2_ref[0].astype(jnp.bfloat16), w1b_ref[...].astype(jnp.bfloat16),
        (((0,), (0,)), ((), ())), preferred_element_type=jnp.float32)
    y = y + b1_ref[...]
    y_ref[...] = y.astype(jnp.bfloat16)

    @pl.when((pl.program_id(0) == 0) & (pl.program_id(1) == 0))
    def _():
        s_ref[...] = jnp.zeros_like(s_ref)
        q_ref[...] = jnp.zeros_like(q_ref)

    s_ref[...] += jnp.sum(y, axis=0, keepdims=True)
    q_ref[...] += jnp.sum(y * y, axis=0, keepdims=True)


def _mlp2_body(y1_ref, s1_ref, q1_ref, g1_ref, bt1_ref, w2_ref, b2_ref,
               y2_ref, s_ref, q_ref):
    mean1 = s1_ref[...] * (1.0 / NQ)
    var1 = q1_ref[...] * (1.0 / NQ) - mean1 * mean1
    a1 = g1_ref[...] / jnp.sqrt(var1 + EPS)
    c1 = bt1_ref[...] - mean1 * a1
    y1 = y1_ref[...].astype(jnp.float32)
    h = jnp.maximum(y1 * a1 + c1, 0.0)
    y = jnp.dot(h.astype(jnp.bfloat16), w2_ref[...].astype(jnp.bfloat16),
                preferred_element_type=jnp.float32)
    y = y + b2_ref[...]
    y2_ref[...] = y.astype(jnp.bfloat16)

    @pl.when(pl.program_id(0) == 0)
    def _():
        s_ref[...] = jnp.zeros_like(s_ref)
        q_ref[...] = jnp.zeros_like(q_ref)

    s_ref[...] += jnp.sum(y, axis=0, keepdims=True)
    q_ref[...] += jnp.sum(y * y, axis=0, keepdims=True)


def _fin_body(y2_ref, s2_ref, q2_ref, g2_ref, bt2_ref, o_ref):
    mean2 = s2_ref[...] * (1.0 / NQ)
    var2 = q2_ref[...] * (1.0 / NQ) - mean2 * mean2
    a2 = g2_ref[...] / jnp.sqrt(var2 + EPS)
    c2 = bt2_ref[...] - mean2 * a2
    y2 = y2_ref[...].astype(jnp.float32)
    h = jnp.maximum(y2 * a2 + c2, 0.0)
    o_ref[0] = h.T


def kernel(points1, points2, features1, features2, W1, b1, gamma1, beta1,
           W2, b2, gamma2, beta2):
    p2t = jnp.transpose(points2, (0, 2, 1))
    w1aT = jnp.transpose(W1[:, :C1])
    w1bT = jnp.transpose(W1[:, C1:])
    w2T = jnp.transpose(W2)
    b1r = b1.reshape(1, O1)

    g = pl.pallas_call(
        _gtable_body,
        grid=(B,),
        in_specs=[
            pl.BlockSpec((1, C1, N1), lambda bb: (bb, 0, 0)),
            pl.BlockSpec((C1, O1), lambda bb: (0, 0)),
        ],
        out_specs=pl.BlockSpec((N1, O1), lambda bb: (bb, 0)),
        out_shape=jax.ShapeDtypeStruct((B * N1, O1), jnp.float32),
    )(features1, w1aT)

    nn = pl.pallas_call(
        _nn3_body,
        grid=(B, N2 // QT),
        in_specs=[
            pl.BlockSpec((1, QT, 3), lambda bb, t: (bb, t, 0)),
            pl.BlockSpec((1, 3, N1), lambda bb, t: (bb, 0, 0)),
        ],
        out_specs=(
            [pl.BlockSpec((1, QT, 1), lambda bb, t: (bb, t, 0))] * 3
            + [pl.BlockSpec((1, QT, 48), lambda bb, t: (bb, t, 0))]),
        out_shape=(
            [jax.ShapeDtypeStruct((B, N2, 1), jnp.int32)] * 3
            + [jax.ShapeDtypeStruct((B, N2, 48), jnp.float32)]),
    )(p2t, points1)
    i0, i1, i2 = (x.reshape(NQ) for x in nn[:3])
    wexp = nn[3].reshape(NQ * 48)

    mesh = plsc.VectorSubcoreMesh(core_axis_name="c", subcore_axis_name="s")
    interp = functools.partial(
        pl.kernel,
        mesh=mesh,
        out_type=jax.ShapeDtypeStruct((NQ, O1), jnp.float32),
        scratch_types=(
            [pltpu.VMEM((QPW,), jnp.int32)] * 3
            + ([pltpu.VMEM((QCH, O1), jnp.float32)] * 3
               + [pltpu.VMEM((QCH * 48,), jnp.float32)]
               + [pltpu.VMEM((QCH, O1), jnp.float32)]) * 2
            + [pltpu.SemaphoreType.DMA] * 4
        ),
    )(_interp_body)
    y1a = interp(g, i0, i1, i2, wexp)

    y1raw, s1, q1 = pl.pallas_call(
        _mlp1_body,
        grid=(B, N2 // RT),
        in_specs=[
            pl.BlockSpec((RT, O1), lambda bb, t: (bb * (N2 // RT) + t, 0)),
            pl.BlockSpec((1, C2, RT), lambda bb, t: (bb, 0, t)),
            pl.BlockSpec((C2, O1), lambda bb, t: (0, 0)),
            pl.BlockSpec((1, O1), lambda bb, t: (0, 0)),
        ],
        out_specs=[
            pl.BlockSpec((RT, O1), lambda bb, t: (bb * (N2 // RT) + t, 0)),
            pl.BlockSpec((1, O1), lambda bb, t: (0, 0)),
            pl.BlockSpec((1, O1), lambda bb, t: (0, 0)),
        ],
        out_shape=[
            jax.ShapeDtypeStruct((NQ, O1), jnp.bfloat16),
            jax.ShapeDtypeStruct((1, O1), jnp.float32),
            jax.ShapeDtypeStruct((1, O1), jnp.float32),
        ],
    )(y1a, features2, w1bT, b1r)

    y2raw, s2, q2 = pl.pallas_call(
        _mlp2_body,
        grid=(NQ // RT,),
        in_specs=[
            pl.BlockSpec((RT, O1), lambda i: (i, 0)),
            pl.BlockSpec((1, O1), lambda i: (0, 0)),
            pl.BlockSpec((1, O1), lambda i: (0, 0)),
            pl.BlockSpec((1, O1), lambda i: (0, 0)),
            pl.BlockSpec((1, O1), lambda i: (0, 0)),
            pl.BlockSpec((O1, O2), lambda i: (0, 0)),
            pl.BlockSpec((1, O2), lambda i: (0, 0)),
        ],
        out_specs=[
            pl.BlockSpec((RT, O2), lambda i: (i, 0)),
            pl.BlockSpec((1, O2), lambda i: (0, 0)),
            pl.BlockSpec((1, O2), lambda i: (0, 0)),
        ],
        out_shape=[
            jax.ShapeDtypeStruct((NQ, O2), jnp.bfloat16),
            jax.ShapeDtypeStruct((1, O2), jnp.float32),
            jax.ShapeDtypeStruct((1, O2), jnp.float32),
        ],
    )(y1raw, s1, q1, gamma1.reshape(1, O1), beta1.reshape(1, O1),
      w2T, b2.reshape(1, O2))

    out = pl.pallas_call(
        _fin_body,
        grid=(B, N2 // NT2),
        in_specs=[
            pl.BlockSpec((NT2, O2), lambda bb, t: (bb * (N2 // NT2) + t, 0)),
            pl.BlockSpec((1, O2), lambda bb, t: (0, 0)),
            pl.BlockSpec((1, O2), lambda bb, t: (0, 0)),
            pl.BlockSpec((1, O2), lambda bb, t: (0, 0)),
            pl.BlockSpec((1, O2), lambda bb, t: (0, 0)),
        ],
        out_specs=pl.BlockSpec((1, O2, NT2), lambda bb, t: (bb, 0, t)),
        out_shape=jax.ShapeDtypeStruct((B, O2, N2), jnp.float32),
    )(y2raw, s2, q2, gamma2.reshape(1, O2), beta2.reshape(1, O2))
    return out


# speedup vs baseline: 15.6477x; 1.0849x over previous
import functools

import jax
import jax.numpy as jnp
from jax import lax
from jax.experimental import pallas as pl
from jax.experimental.pallas import tpu as pltpu
from jax.experimental.pallas import tpu_sc as plsc

B, N1, N2 = 16, 1024, 4096
C1, C2 = 512, 256
O1, O2 = 512, 256
NQ = B * N2
EPS = 1e-3

BH = B // 2
NQH = BH * N2
QT = 512
RT = 512
NT2 = 256
NW = 32
QPW = NQH // NW
QCH = 16
NCH = QPW // QCH


def _gtable_body(f1_ref, w_ref, g_ref):
    g_ref[...] = lax.dot_general(
        f1_ref[0].astype(jnp.bfloat16), w_ref[...].astype(jnp.bfloat16),
        (((0,), (0,)), ((), ())), preferred_element_type=jnp.float32)


def _nn3_body(p2_ref, p1_ref, i0_ref, i1_ref, i2_ref, wexp_ref, *, boff):
    b = pl.program_id(0)
    q = p2_ref[0]
    kpts = p1_ref[0]
    d2 = jnp.zeros((QT, N1), jnp.float32)
    for j in range(3):
        diff = q[:, j:j + 1] - kpts[j:j + 1, :]
        d2 = d2 + diff * diff
    lanef = lax.broadcasted_iota(jnp.int32, (QT, N1), 1).astype(jnp.float32)
    cur = d2
    mins, args = [], []
    for t in range(3):
        m = jnp.min(cur, axis=1, keepdims=True)
        amf = jnp.min(jnp.where(cur == m, lanef, jnp.float32(2.0 ** 30)),
                      axis=1, keepdims=True)
        if t < 2:
            cur = jnp.where(lanef == amf, jnp.float32(jnp.inf), cur)
        mins.append(m)
        args.append(amf.astype(jnp.int32))
    invs = []
    for m in mins:
        d = jnp.sqrt(jnp.maximum(m, 0.0))
        dd = d * d
        dd = jnp.where(dd < 1e-10, jnp.float32(1e-10), dd)
        invs.append(1.0 / dd)
    norm = (invs[0] + invs[1]) + invs[2]
    base = (b + boff) * N1
    i0_ref[0] = args[0] + base
    i1_ref[0] = args[1] + base
    i2_ref[0] = args[2] + base
    wexp_ref[0] = jnp.concatenate(
        [jnp.broadcast_to(iv / norm, (QT, 16)) for iv in invs], axis=1)


def _interp_body(g_hbm, i0_hbm, i1_hbm, i2_hbm, wexp_hbm, out_hbm,
                 i0v, i1v, i2v,
                 ra0, ra1, ra2, wva, ova,
                 rb0, rb1, rb2, wvb, ovb,
                 gsa, gsb, osa, osb):
    cid = lax.axis_index("c")
    sid = lax.axis_index("s")
    wid = sid * 2 + cid
    base = wid * QPW
    pltpu.sync_copy(i0_hbm.at[pl.ds(base, QPW)], i0v)
    pltpu.sync_copy(i1_hbm.at[pl.ds(base, QPW)], i1v)
    pltpu.sync_copy(i2_hbm.at[pl.ds(base, QPW)], i2v)

    def fire(ci, rr0, rr1, rr2, wv, gs):
        cb = ci * QCH
        pltpu.async_copy(g_hbm.at[i0v.at[pl.ds(cb, QCH)]], rr0, gs)
        pltpu.async_copy(g_hbm.at[i1v.at[pl.ds(cb, QCH)]], rr1, gs)
        pltpu.async_copy(g_hbm.at[i2v.at[pl.ds(cb, QCH)]], rr2, gs)
        pltpu.async_copy(wexp_hbm.at[pl.ds((base + cb) * 48, QCH * 48)],
                         wv, gs)

    def drain_gather(rr0, rr1, rr2, wv, gs):
        pltpu.make_async_copy(g_hbm.at[pl.ds(0, QCH)], rr0, gs).wait()
        pltpu.make_async_copy(g_hbm.at[pl.ds(0, QCH)], rr1, gs).wait()
        pltpu.make_async_copy(g_hbm.at[pl.ds(0, QCH)], rr2, gs).wait()
        pltpu.make_async_copy(wexp_hbm.at[pl.ds(0, QCH * 48)], wv, gs).wait()

    def process(ci, rr0, rr1, rr2, wv, ov, gs, os,
                nr0, nr1, nr2, nwv, ngs):
        @pl.when(ci + 1 < NCH)
        def _():
            fire(ci + 1, nr0, nr1, nr2, nwv, ngs)

        drain_gather(rr0, rr1, rr2, wv, gs)

        @pl.when(ci >= 2)
        def _():
            pltpu.make_async_copy(ov, out_hbm.at[pl.ds(0, QCH)], os).wait()

        def qloop(qi, carry2):
            s0 = wv[pl.ds(qi * 48, 16)]
            s1 = wv[pl.ds(qi * 48 + 16, 16)]
            s2 = wv[pl.ds(qi * 48 + 32, 16)]
            for ch in range(O1 // 16):
                sl = pl.ds(ch * 16, 16)
                ov[qi, sl] = (s0 * rr0[qi, sl] + s1 * rr1[qi, sl]) \
                    + s2 * rr2[qi, sl]
            return carry2

        lax.fori_loop(0, QCH, qloop, 0)
        pltpu.async_copy(ov, out_hbm.at[pl.ds(base + ci * QCH, QCH)], os)

    fire(0, ra0, ra1, ra2, wva, gsa)

    def chunk(ci, carry):
        even = lax.rem(ci, 2) == 0

        @pl.when(even)
        def _():
            process(ci, ra0, ra1, ra2, wva, ova, gsa, osa,
                    rb0, rb1, rb2, wvb, gsb)

        @pl.when(jnp.logical_not(even))
        def _():
            process(ci, rb0, rb1, rb2, wvb, ovb, gsb, osb,
                    ra0, ra1, ra2, wva, gsa)

        return carry

    lax.fori_loop(0, NCH, chunk, 0)
    pltpu.make_async_copy(ova, out_hbm.at[pl.ds(0, QCH)], osa).wait()
    pltpu.make_async_copy(ovb, out_hbm.at[pl.ds(0, QCH)], osb).wait()


def _mlp1_body(y1a_ref, f2_ref, w1b_ref, b1_ref, y_ref, s_ref, q_ref):
    y = y1a_ref[...] + lax.dot_general(
        f2_ref[0].astype(jnp.bfloat16), w1b_ref[...].astype(jnp.bfloat16),
        (((0,), (0,)), ((), ())), preferred_element_type=jnp.float32)
    y = y + b1_ref[...]
    y_ref[...] = y.astype(jnp.bfloat16)

    @pl.when((pl.program_id(0) == 0) & (pl.program_id(1) == 0))
    def _():
        s_ref[...] = jnp.zeros_like(s_ref)
        q_ref[...] = jnp.zeros_like(q_ref)

    s_ref[...] += jnp.sum(y, axis=0, keepdims=True)
    q_ref[...] += jnp.sum(y * y, axis=0, keepdims=True)


def _mlp2_body(y1_ref, s1a_ref, q1a_ref, s1b_ref, q1b_ref, g1_ref, bt1_ref,
               w2_ref, b2_ref, y2_ref, s_ref, q_ref):
    mean1 = (s1a_ref[...] + s1b_ref[...]) * (1.0 / NQ)
    var1 = (q1a_ref[...] + q1b_ref[...]) * (1.0 / NQ) - mean1 * mean1
    a1 = g1_ref[...] / jnp.sqrt(var1 + EPS)
    c1 = bt1_ref[...] - mean1 * a1
    y1 = y1_ref[...].astype(jnp.float32)
    h = jnp.maximum(y1 * a1 + c1, 0.0)
    y = jnp.dot(h.astype(jnp.bfloat16), w2_ref[...].astype(jnp.bfloat16),
                preferred_element_type=jnp.float32)
    y = y + b2_ref[...]
    y2_ref[...] = y.astype(jnp.bfloat16)

    @pl.when(pl.program_id(0) == 0)
    def _():
        s_ref[...] = jnp.zeros_like(s_ref)
        q_ref[...] = jnp.zeros_like(q_ref)

    s_ref[...] += jnp.sum(y, axis=0, keepdims=True)
    q_ref[...] += jnp.sum(y * y, axis=0, keepdims=True)


def _fin_body(y2_ref, s2a_ref, q2a_ref, s2b_ref, q2b_ref, g2_ref, bt2_ref,
              o_ref):
    mean2 = (s2a_ref[...] + s2b_ref[...]) * (1.0 / NQ)
    var2 = (q2a_ref[...] + q2b_ref[...]) * (1.0 / NQ) - mean2 * mean2
    a2 = g2_ref[...] / jnp.sqrt(var2 + EPS)
    c2 = bt2_ref[...] - mean2 * a2
    y2 = y2_ref[...].astype(jnp.float32)
    h = jnp.maximum(y2 * a2 + c2, 0.0)
    o_ref[0] = h.T


def _nn3_half(p2t, points1, boff):
    return pl.pallas_call(
        functools.partial(_nn3_body, boff=boff),
        grid=(BH, N2 // QT),
        in_specs=[
            pl.BlockSpec((1, QT, 3), lambda bb, t, bo=boff: (bb + bo, t, 0)),
            pl.BlockSpec((1, 3, N1), lambda bb, t, bo=boff: (bb + bo, 0, 0)),
        ],
        out_specs=(
            [pl.BlockSpec((1, QT, 1), lambda bb, t: (bb, t, 0))] * 3
            + [pl.BlockSpec((1, QT, 48), lambda bb, t: (bb, t, 0))]),
        out_shape=(
            [jax.ShapeDtypeStruct((BH, N2, 1), jnp.int32)] * 3
            + [jax.ShapeDtypeStruct((BH, N2, 48), jnp.float32)]),
    )(p2t, points1)


def _interp_half(g, nn):
    i0, i1, i2 = (x.reshape(NQH) for x in nn[:3])
    wexp = nn[3].reshape(NQH * 48)
    mesh = plsc.VectorSubcoreMesh(core_axis_name="c", subcore_axis_name="s")
    interp = functools.partial(
        pl.kernel,
        mesh=mesh,
        out_type=jax.ShapeDtypeStruct((NQH, O1), jnp.float32),
        scratch_types=(
            [pltpu.VMEM((QPW,), jnp.int32)] * 3
            + ([pltpu.VMEM((QCH, O1), jnp.float32)] * 3
               + [pltpu.VMEM((QCH * 48,), jnp.float32)]
               + [pltpu.VMEM((QCH, O1), jnp.float32)]) * 2
            + [pltpu.SemaphoreType.DMA] * 4
        ),
    )(_interp_body)
    return interp(g, i0, i1, i2, wexp)


def _mlp1_half(y1a, features2, w1bT, b1r, boff):
    return pl.pallas_call(
        _mlp1_body,
        grid=(BH, N2 // RT),
        in_specs=[
            pl.BlockSpec((RT, O1), lambda bb, t: (bb * (N2 // RT) + t, 0)),
            pl.BlockSpec((1, C2, RT), lambda bb, t, bo=boff: (bb + bo, 0, t)),
            pl.BlockSpec((C2, O1), lambda bb, t: (0, 0)),
            pl.BlockSpec((1, O1), lambda bb, t: (0, 0)),
        ],
        out_specs=[
            pl.BlockSpec((RT, O1), lambda bb, t: (bb * (N2 // RT) + t, 0)),
            pl.BlockSpec((1, O1), lambda bb, t: (0, 0)),
            pl.BlockSpec((1, O1), lambda bb, t: (0, 0)),
        ],
        out_shape=[
            jax.ShapeDtypeStruct((NQH, O1), jnp.bfloat16),
            jax.ShapeDtypeStruct((1, O1), jnp.float32),
            jax.ShapeDtypeStruct((1, O1), jnp.float32),
        ],
    )(y1a, features2, w1bT, b1r)


def _mlp2_half(y1raw, stats1, g1r, bt1r, w2T, b2r):
    return pl.pallas_call(
        _mlp2_body,
        grid=(NQH // RT,),
        in_specs=[
            pl.BlockSpec((RT, O1), lambda i: (i, 0)),
            pl.BlockSpec((1, O1), lambda i: (0, 0)),
            pl.BlockSpec((1, O1), lambda i: (0, 0)),
            pl.BlockSpec((1, O1), lambda i: (0, 0)),
            pl.BlockSpec((1, O1), lambda i: (0, 0)),
            pl.BlockSpec((1, O1), lambda i: (0, 0)),
            pl.BlockSpec((1, O1), lambda i: (0, 0)),
            pl.BlockSpec((O1, O2), lambda i: (0, 0)),
            pl.BlockSpec((1, O2), lambda i: (0, 0)),
        ],
        out_specs=[
            pl.BlockSpec((RT, O2), lambda i: (i, 0)),
            pl.BlockSpec((1, O2), lambda i: (0, 0)),
            pl.BlockSpec((1, O2), lambda i: (0, 0)),
        ],
        out_shape=[
            jax.ShapeDtypeStruct((NQH, O2), jnp.bfloat16),
            jax.ShapeDtypeStruct((1, O2), jnp.float32),
            jax.ShapeDtypeStruct((1, O2), jnp.float32),
        ],
    )(y1raw, *stats1, g1r, bt1r, w2T, b2r)


def _fin_half(y2raw, stats2, g2r, bt2r):
    return pl.pallas_call(
        _fin_body,
        grid=(BH, N2 // NT2),
        in_specs=[
            pl.BlockSpec((NT2, O2), lambda bb, t: (bb * (N2 // NT2) + t, 0)),
            pl.BlockSpec((1, O2), lambda bb, t: (0, 0)),
            pl.BlockSpec((1, O2), lambda bb, t: (0, 0)),
            pl.BlockSpec((1, O2), lambda bb, t: (0, 0)),
            pl.BlockSpec((1, O2), lambda bb, t: (0, 0)),
            pl.BlockSpec((1, O2), lambda bb, t: (0, 0)),
            pl.BlockSpec((1, O2), lambda bb, t: (0, 0)),
        ],
        out_specs=pl.BlockSpec((1, O2, NT2), lambda bb, t: (bb, 0, t)),
        out_shape=jax.ShapeDtypeStruct((BH, O2, N2), jnp.float32),
    )(y2raw, *stats2, g2r, bt2r)


def kernel(points1, points2, features1, features2, W1, b1, gamma1, beta1,
           W2, b2, gamma2, beta2):
    p2t = jnp.transpose(points2, (0, 2, 1))
    w1aT = jnp.transpose(W1[:, :C1])
    w1bT = jnp.transpose(W1[:, C1:])
    w2T = jnp.transpose(W2)
    b1r = b1.reshape(1, O1)
    b2r = b2.reshape(1, O2)
    g1r = gamma1.reshape(1, O1)
    bt1r = beta1.reshape(1, O1)
    g2r = gamma2.reshape(1, O2)
    bt2r = beta2.reshape(1, O2)

    g = pl.pallas_call(
        _gtable_body,
        grid=(B,),
        in_specs=[
            pl.BlockSpec((1, C1, N1), lambda bb: (bb, 0, 0)),
            pl.BlockSpec((C1, O1), lambda bb: (0, 0)),
        ],
        out_specs=pl.BlockSpec((N1, O1), lambda bb: (bb, 0)),
        out_shape=jax.ShapeDtypeStruct((B * N1, O1), jnp.float32),
    )(features1, w1aT)

    nn_a = _nn3_half(p2t, points1, 0)
    y1a_a = _interp_half(g, nn_a)
    nn_b = _nn3_half(p2t, points1, BH)
    y1a_b = _interp_half(g, nn_b)

    y1raw_a, s1a, q1a = _mlp1_half(y1a_a, features2, w1bT, b1r, 0)
    y1raw_b, s1b, q1b = _mlp1_half(y1a_b, features2, w1bT, b1r, BH)

    stats1 = (s1a, q1a, s1b, q1b)
    y2raw_a, s2a, q2a = _mlp2_half(y1raw_a, stats1, g1r, bt1r, w2T, b2r)
    y2raw_b, s2b, q2b = _mlp2_half(y1raw_b, stats1, g1r, bt1r, w2T, b2r)

    stats2 = (s2a, q2a, s2b, q2b)
    out_a = _fin_half(y2raw_a, stats2, g2r, bt2r)
    out_b = _fin_half(y2raw_b, stats2, g2r, bt2r)
    return jnp.concatenate([out_a, out_b], axis=0)


# speedup vs baseline: 16.3306x; 1.0436x over previous
import functools

import jax
import jax.numpy as jnp
from jax import lax
from jax.experimental import pallas as pl
from jax.experimental.pallas import tpu as pltpu
from jax.experimental.pallas import tpu_sc as plsc

B, N1, N2 = 16, 1024, 4096
C1, C2 = 512, 256
O1, O2 = 512, 256
NQ = B * N2
EPS = 1e-3

BH = B // 2
NQH = BH * N2
QT = 512
RT = 512
NT2 = 256
NW = 32
QPW = NQH // NW
QCH = 16
NCH = QPW // QCH


def _gtable_body(f1_ref, w_ref, g_ref):
    g_ref[...] = lax.dot_general(
        f1_ref[0].astype(jnp.bfloat16), w_ref[...].astype(jnp.bfloat16),
        (((0,), (0,)), ((), ())), preferred_element_type=jnp.float32)


def _nn3_body(p2_ref, p1_ref, i0_ref, i1_ref, i2_ref, wexp_ref, *, boff):
    b = pl.program_id(0)
    q = p2_ref[0]
    kpts = p1_ref[0]
    d2 = jnp.zeros((QT, N1), jnp.float32)
    for j in range(3):
        diff = q[:, j:j + 1] - kpts[j:j + 1, :]
        d2 = d2 + diff * diff
    lanef = lax.broadcasted_iota(jnp.int32, (QT, N1), 1).astype(jnp.float32)
    cur = d2
    mins, args = [], []
    for t in range(3):
        m = jnp.min(cur, axis=1, keepdims=True)
        amf = jnp.min(jnp.where(cur == m, lanef, jnp.float32(2.0 ** 30)),
                      axis=1, keepdims=True)
        if t < 2:
            cur = jnp.where(lanef == amf, jnp.float32(jnp.inf), cur)
        mins.append(m)
        args.append(amf.astype(jnp.int32))
    invs = []
    for m in mins:
        d = jnp.sqrt(jnp.maximum(m, 0.0))
        dd = d * d
        dd = jnp.where(dd < 1e-10, jnp.float32(1e-10), dd)
        invs.append(1.0 / dd)
    norm = (invs[0] + invs[1]) + invs[2]
    base = (b + boff) * N1
    i0_ref[0] = args[0] + base
    i1_ref[0] = args[1] + base
    i2_ref[0] = args[2] + base
    wexp_ref[0] = jnp.concatenate(
        [jnp.broadcast_to(iv / norm, (QT, 16)) for iv in invs], axis=1)


def _interp_body(g_hbm, i0_hbm, i1_hbm, i2_hbm, wexp_hbm, out_hbm,
                 i0v, i1v, i2v,
                 ra0, ra1, ra2, wva, ova,
                 rb0, rb1, rb2, wvb, ovb,
                 gsa, gsb, osa, osb):
    cid = lax.axis_index("c")
    sid = lax.axis_index("s")
    wid = sid * 2 + cid
    base = wid * QPW
    pltpu.sync_copy(i0_hbm.at[pl.ds(base, QPW)], i0v)
    pltpu.sync_copy(i1_hbm.at[pl.ds(base, QPW)], i1v)
    pltpu.sync_copy(i2_hbm.at[pl.ds(base, QPW)], i2v)

    def fire(ci, rr0, rr1, rr2, wv, gs):
        cb = ci * QCH
        pltpu.async_copy(g_hbm.at[i0v.at[pl.ds(cb, QCH)]], rr0, gs)
        pltpu.async_copy(g_hbm.at[i1v.at[pl.ds(cb, QCH)]], rr1, gs)
        pltpu.async_copy(g_hbm.at[i2v.at[pl.ds(cb, QCH)]], rr2, gs)
        pltpu.async_copy(wexp_hbm.at[pl.ds((base + cb) * 48, QCH * 48)],
                         wv, gs)

    def drain_gather(rr0, rr1, rr2, wv, gs):
        pltpu.make_async_copy(g_hbm.at[pl.ds(0, QCH)], rr0, gs).wait()
        pltpu.make_async_copy(g_hbm.at[pl.ds(0, QCH)], rr1, gs).wait()
        pltpu.make_async_copy(g_hbm.at[pl.ds(0, QCH)], rr2, gs).wait()
        pltpu.make_async_copy(wexp_hbm.at[pl.ds(0, QCH * 48)], wv, gs).wait()

    def process(ci, rr0, rr1, rr2, wv, ov, gs, os,
                nr0, nr1, nr2, nwv, ngs):
        @pl.when(ci + 1 < NCH)
        def _():
            fire(ci + 1, nr0, nr1, nr2, nwv, ngs)

        drain_gather(rr0, rr1, rr2, wv, gs)

        @pl.when(ci >= 2)
        def _():
            pltpu.make_async_copy(ov, out_hbm.at[pl.ds(0, QCH)], os).wait()

        def qloop(qi, carry2):
            s0 = wv[pl.ds(qi * 48, 16)]
            s1 = wv[pl.ds(qi * 48 + 16, 16)]
            s2 = wv[pl.ds(qi * 48 + 32, 16)]
            for ch in range(O1 // 16):
                sl = pl.ds(ch * 16, 16)
                ov[qi, sl] = (s0 * rr0[qi, sl] + s1 * rr1[qi, sl]) \
                    + s2 * rr2[qi, sl]
            return carry2

        lax.fori_loop(0, QCH, qloop, 0)
        pltpu.async_copy(ov, out_hbm.at[pl.ds(base + ci * QCH, QCH)], os)

    fire(0, ra0, ra1, ra2, wva, gsa)

    def chunk(ci, carry):
        even = lax.rem(ci, 2) == 0

        @pl.when(even)
        def _():
            process(ci, ra0, ra1, ra2, wva, ova, gsa, osa,
                    rb0, rb1, rb2, wvb, gsb)

        @pl.when(jnp.logical_not(even))
        def _():
            process(ci, rb0, rb1, rb2, wvb, ovb, gsb, osb,
                    ra0, ra1, ra2, wva, gsa)

        return carry

    lax.fori_loop(0, NCH, chunk, 0)
    pltpu.make_async_copy(ova, out_hbm.at[pl.ds(0, QCH)], osa).wait()
    pltpu.make_async_copy(ovb, out_hbm.at[pl.ds(0, QCH)], osb).wait()


def _mlp1_body(y1a_ref, f2_ref, w1b_ref, b1_ref, y_ref, s_ref, q_ref):
    y = y1a_ref[...] + lax.dot_general(
        f2_ref[0].astype(jnp.bfloat16), w1b_ref[...].astype(jnp.bfloat16),
        (((0,), (0,)), ((), ())), preferred_element_type=jnp.float32)
    y = y + b1_ref[...]
    y_ref[...] = y.astype(jnp.bfloat16)

    @pl.when((pl.program_id(0) == 0) & (pl.program_id(1) == 0))
    def _():
        s_ref[...] = jnp.zeros_like(s_ref)
        q_ref[...] = jnp.zeros_like(q_ref)

    s_ref[...] += jnp.sum(y, axis=0, keepdims=True)
    q_ref[...] += jnp.sum(y * y, axis=0, keepdims=True)


def _mlp2_body(y1_ref, s1a_ref, q1a_ref, s1b_ref, q1b_ref, g1_ref, bt1_ref,
               w2_ref, b2_ref, y2_ref, s_ref, q_ref):
    mean1 = (s1a_ref[...] + s1b_ref[...]) * (1.0 / NQ)
    var1 = (q1a_ref[...] + q1b_ref[...]) * (1.0 / NQ) - mean1 * mean1
    a1 = g1_ref[...] / jnp.sqrt(var1 + EPS)
    c1 = bt1_ref[...] - mean1 * a1
    y1 = y1_ref[...].astype(jnp.float32)
    h = jnp.maximum(y1 * a1 + c1, 0.0)
    y = jnp.dot(h.astype(jnp.bfloat16), w2_ref[...].astype(jnp.bfloat16),
                preferred_element_type=jnp.float32)
    y = y + b2_ref[...]
    y2_ref[...] = y.astype(jnp.bfloat16)

    @pl.when(pl.program_id(0) == 0)
    def _():
        s_ref[...] = jnp.zeros_like(s_ref)
        q_ref[...] = jnp.zeros_like(q_ref)

    s_ref[...] += jnp.sum(y, axis=0, keepdims=True)
    q_ref[...] += jnp.sum(y * y, axis=0, keepdims=True)


def _fin_body(y2_ref, s2a_ref, q2a_ref, s2b_ref, q2b_ref, g2_ref, bt2_ref,
              o_ref):
    mean2 = (s2a_ref[...] + s2b_ref[...]) * (1.0 / NQ)
    var2 = (q2a_ref[...] + q2b_ref[...]) * (1.0 / NQ) - mean2 * mean2
    a2 = g2_ref[...] / jnp.sqrt(var2 + EPS)
    c2 = bt2_ref[...] - mean2 * a2
    y2 = y2_ref[...].astype(jnp.float32)
    h = jnp.maximum(y2 * a2 + c2, 0.0)
    o_ref[0] = h.T


def _fin_body_b(prev_ref, y2_ref, s2a_ref, q2a_ref, s2b_ref, q2b_ref,
                g2_ref, bt2_ref, o_ref):
    del prev_ref
    _fin_body(y2_ref, s2a_ref, q2a_ref, s2b_ref, q2b_ref, g2_ref, bt2_ref,
              o_ref)


def _nn3_half(p2t, points1, boff):
    return pl.pallas_call(
        functools.partial(_nn3_body, boff=boff),
        grid=(BH, N2 // QT),
        in_specs=[
            pl.BlockSpec((1, QT, 3), lambda bb, t, bo=boff: (bb + bo, t, 0)),
            pl.BlockSpec((1, 3, N1), lambda bb, t, bo=boff: (bb + bo, 0, 0)),
        ],
        out_specs=(
            [pl.BlockSpec((1, QT, 1), lambda bb, t: (bb, t, 0))] * 3
            + [pl.BlockSpec((1, QT, 48), lambda bb, t: (bb, t, 0))]),
        out_shape=(
            [jax.ShapeDtypeStruct((BH, N2, 1), jnp.int32)] * 3
            + [jax.ShapeDtypeStruct((BH, N2, 48), jnp.float32)]),
    )(p2t, points1)


def _interp_half(g, nn):
    i0, i1, i2 = (x.reshape(NQH) for x in nn[:3])
    wexp = nn[3].reshape(NQH * 48)
    mesh = plsc.VectorSubcoreMesh(core_axis_name="c", subcore_axis_name="s")
    interp = functools.partial(
        pl.kernel,
        mesh=mesh,
        out_type=jax.ShapeDtypeStruct((NQH, O1), jnp.float32),
        scratch_types=(
            [pltpu.VMEM((QPW,), jnp.int32)] * 3
            + ([pltpu.VMEM((QCH, O1), jnp.float32)] * 3
               + [pltpu.VMEM((QCH * 48,), jnp.float32)]
               + [pltpu.VMEM((QCH, O1), jnp.float32)]) * 2
            + [pltpu.SemaphoreType.DMA] * 4
        ),
    )(_interp_body)
    return interp(g, i0, i1, i2, wexp)


def _mlp1_half(y1a, features2, w1bT, b1r, boff):
    return pl.pallas_call(
        _mlp1_body,
        grid=(BH, N2 // RT),
        in_specs=[
            pl.BlockSpec((RT, O1), lambda bb, t: (bb * (N2 // RT) + t, 0)),
            pl.BlockSpec((1, C2, RT), lambda bb, t, bo=boff: (bb + bo, 0, t)),
            pl.BlockSpec((C2, O1), lambda bb, t: (0, 0)),
            pl.BlockSpec((1, O1), lambda bb, t: (0, 0)),
        ],
        out_specs=[
            pl.BlockSpec((RT, O1), lambda bb, t: (bb * (N2 // RT) + t, 0)),
            pl.BlockSpec((1, O1), lambda bb, t: (0, 0)),
            pl.BlockSpec((1, O1), lambda bb, t: (0, 0)),
        ],
        out_shape=[
            jax.ShapeDtypeStruct((NQH, O1), jnp.bfloat16),
            jax.ShapeDtypeStruct((1, O1), jnp.float32),
            jax.ShapeDtypeStruct((1, O1), jnp.float32),
        ],
    )(y1a, features2, w1bT, b1r)


def _mlp2_half(y1raw, stats1, g1r, bt1r, w2T, b2r):
    return pl.pallas_call(
        _mlp2_body,
        grid=(NQH // RT,),
        in_specs=[
            pl.BlockSpec((RT, O1), lambda i: (i, 0)),
            pl.BlockSpec((1, O1), lambda i: (0, 0)),
            pl.BlockSpec((1, O1), lambda i: (0, 0)),
            pl.BlockSpec((1, O1), lambda i: (0, 0)),
            pl.BlockSpec((1, O1), lambda i: (0, 0)),
            pl.BlockSpec((1, O1), lambda i: (0, 0)),
            pl.BlockSpec((1, O1), lambda i: (0, 0)),
            pl.BlockSpec((O1, O2), lambda i: (0, 0)),
            pl.BlockSpec((1, O2), lambda i: (0, 0)),
        ],
        out_specs=[
            pl.BlockSpec((RT, O2), lambda i: (i, 0)),
            pl.BlockSpec((1, O2), lambda i: (0, 0)),
            pl.BlockSpec((1, O2), lambda i: (0, 0)),
        ],
        out_shape=[
            jax.ShapeDtypeStruct((NQH, O2), jnp.bfloat16),
            jax.ShapeDtypeStruct((1, O2), jnp.float32),
            jax.ShapeDtypeStruct((1, O2), jnp.float32),
        ],
    )(y1raw, *stats1, g1r, bt1r, w2T, b2r)


_FIN_STATS_SPECS = [pl.BlockSpec((1, O2), lambda bb, t: (0, 0))] * 6


def _fin_half_a(y2raw, stats2, g2r, bt2r):
    return pl.pallas_call(
        _fin_body,
        grid=(BH, N2 // NT2),
        in_specs=[
            pl.BlockSpec((NT2, O2), lambda bb, t: (bb * (N2 // NT2) + t, 0)),
        ] + _FIN_STATS_SPECS,
        out_specs=pl.BlockSpec((1, O2, NT2), lambda bb, t: (bb, 0, t)),
        out_shape=jax.ShapeDtypeStruct((B, O2, N2), jnp.float32),
    )(y2raw, *stats2, g2r, bt2r)


def _fin_half_b(out_a, y2raw, stats2, g2r, bt2r):
    return pl.pallas_call(
        _fin_body_b,
        grid=(BH, N2 // NT2),
        in_specs=[
            pl.BlockSpec((1, O2, NT2), lambda bb, t: (0, 0, 0)),
            pl.BlockSpec((NT2, O2), lambda bb, t: (bb * (N2 // NT2) + t, 0)),
        ] + _FIN_STATS_SPECS,
        out_specs=pl.BlockSpec((1, O2, NT2), lambda bb, t: (bb + BH, 0, t)),
        out_shape=jax.ShapeDtypeStruct((B, O2, N2), jnp.float32),
        input_output_aliases={0: 0},
    )(out_a, y2raw, *stats2, g2r, bt2r)


def kernel(points1, points2, features1, features2, W1, b1, gamma1, beta1,
           W2, b2, gamma2, beta2):
    p2t = jnp.transpose(points2, (0, 2, 1))
    w1aT = jnp.transpose(W1[:, :C1])
    w1bT = jnp.transpose(W1[:, C1:])
    w2T = jnp.transpose(W2)
    b1r = b1.reshape(1, O1)
    b2r = b2.reshape(1, O2)
    g1r = gamma1.reshape(1, O1)
    bt1r = beta1.reshape(1, O1)
    g2r = gamma2.reshape(1, O2)
    bt2r = beta2.reshape(1, O2)

    g = pl.pallas_call(
        _gtable_body,
        grid=(B,),
        in_specs=[
            pl.BlockSpec((1, C1, N1), lambda bb: (bb, 0, 0)),
            pl.BlockSpec((C1, O1), lambda bb: (0, 0)),
        ],
        out_specs=pl.BlockSpec((N1, O1), lambda bb: (bb, 0)),
        out_shape=jax.ShapeDtypeStruct((B * N1, O1), jnp.float32),
    )(features1, w1aT)

    nn_a = _nn3_half(p2t, points1, 0)
    y1a_a = _interp_half(g, nn_a)
    nn_b = _nn3_half(p2t, points1, BH)
    y1a_b = _interp_half(g, nn_b)

    y1raw_a, s1a, q1a = _mlp1_half(y1a_a, features2, w1bT, b1r, 0)
    y1raw_b, s1b, q1b = _mlp1_half(y1a_b, features2, w1bT, b1r, BH)

    stats1 = (s1a, q1a, s1b, q1b)
    y2raw_a, s2a, q2a = _mlp2_half(y1raw_a, stats1, g1r, bt1r, w2T, b2r)
    y2raw_b, s2b, q2b = _mlp2_half(y1raw_b, stats1, g1r, bt1r, w2T, b2r)

    stats2 = (s2a, q2a, s2b, q2b)
    out_a = _fin_half_a(y2raw_a, stats2, g2r, bt2r)
    return _fin_half_b(out_a, y2raw_b, stats2, g2r, bt2r)


# speedup vs baseline: 16.3530x; 1.0014x over previous
import functools

import jax
import jax.numpy as jnp
from jax import lax
from jax.experimental import pallas as pl
from jax.experimental.pallas import tpu as pltpu
from jax.experimental.pallas import tpu_sc as plsc

B, N1, N2 = 16, 1024, 4096
C1, C2 = 512, 256
O1, O2 = 512, 256
NQ = B * N2
EPS = 1e-3

BH = B // 2
NQH = BH * N2
QT = 512
RT = 512
NT2 = 256
NW = 32
QPW = NQH // NW
QCH = 16
NCH = QPW // QCH


def _gtable_body(f1_ref, w_ref, g_ref):
    g_ref[...] = lax.dot_general(
        f1_ref[0].astype(jnp.bfloat16), w_ref[...].astype(jnp.bfloat16),
        (((0,), (0,)), ((), ())), preferred_element_type=jnp.float32)


def _nn3_body(p2_ref, p1_ref, f1_ref, w_ref,
              i0_ref, i1_ref, i2_ref, wexp_ref, g_ref, *, boff):
    b = pl.program_id(0)

    @pl.when(pl.program_id(1) == 0)
    def _():
        g_ref[...] = lax.dot_general(
            f1_ref[0].astype(jnp.bfloat16), w_ref[...].astype(jnp.bfloat16),
            (((0,), (0,)), ((), ())), preferred_element_type=jnp.float32)

    q = p2_ref[0]
    kpts = p1_ref[0]
    d2 = jnp.zeros((QT, N1), jnp.float32)
    for j in range(3):
        diff = q[:, j:j + 1] - kpts[j:j + 1, :]
        d2 = d2 + diff * diff
    lanef = lax.broadcasted_iota(jnp.int32, (QT, N1), 1).astype(jnp.float32)
    cur = d2
    mins, args = [], []
    for t in range(3):
        m = jnp.min(cur, axis=1, keepdims=True)
        amf = jnp.min(jnp.where(cur == m, lanef, jnp.float32(2.0 ** 30)),
                      axis=1, keepdims=True)
        if t < 2:
            cur = jnp.where(lanef == amf, jnp.float32(jnp.inf), cur)
        mins.append(m)
        args.append(amf.astype(jnp.int32))
    invs = []
    for m in mins:
        d = jnp.sqrt(jnp.maximum(m, 0.0))
        dd = d * d
        dd = jnp.where(dd < 1e-10, jnp.float32(1e-10), dd)
        invs.append(1.0 / dd)
    norm = (invs[0] + invs[1]) + invs[2]
    base = (b + boff) * N1
    i0_ref[0] = args[0] + base
    i1_ref[0] = args[1] + base
    i2_ref[0] = args[2] + base
    wexp_ref[0] = jnp.concatenate(
        [jnp.broadcast_to(iv / norm, (QT, 16)) for iv in invs], axis=1)


def _interp_body(g_hbm, i0_hbm, i1_hbm, i2_hbm, wexp_hbm, out_hbm,
                 i0v, i1v, i2v,
                 ra0, ra1, ra2, wva, ova,
                 rb0, rb1, rb2, wvb, ovb,
                 gsa, gsb, osa, osb):
    cid = lax.axis_index("c")
    sid = lax.axis_index("s")
    wid = sid * 2 + cid
    base = wid * QPW
    pltpu.sync_copy(i0_hbm.at[pl.ds(base, QPW)], i0v)
    pltpu.sync_copy(i1_hbm.at[pl.ds(base, QPW)], i1v)
    pltpu.sync_copy(i2_hbm.at[pl.ds(base, QPW)], i2v)

    def fire(ci, rr0, rr1, rr2, wv, gs):
        cb = ci * QCH
        pltpu.async_copy(g_hbm.at[i0v.at[pl.ds(cb, QCH)]], rr0, gs)
        pltpu.async_copy(g_hbm.at[i1v.at[pl.ds(cb, QCH)]], rr1, gs)
        pltpu.async_copy(g_hbm.at[i2v.at[pl.ds(cb, QCH)]], rr2, gs)
        pltpu.async_copy(wexp_hbm.at[pl.ds((base + cb) * 48, QCH * 48)],
                         wv, gs)

    def drain_gather(rr0, rr1, rr2, wv, gs):
        pltpu.make_async_copy(g_hbm.at[pl.ds(0, QCH)], rr0, gs).wait()
        pltpu.make_async_copy(g_hbm.at[pl.ds(0, QCH)], rr1, gs).wait()
        pltpu.make_async_copy(g_hbm.at[pl.ds(0, QCH)], rr2, gs).wait()
        pltpu.make_async_copy(wexp_hbm.at[pl.ds(0, QCH * 48)], wv, gs).wait()

    def process(ci, rr0, rr1, rr2, wv, ov, gs, os,
                nr0, nr1, nr2, nwv, ngs):
        @pl.when(ci + 1 < NCH)
        def _():
            fire(ci + 1, nr0, nr1, nr2, nwv, ngs)

        drain_gather(rr0, rr1, rr2, wv, gs)

        @pl.when(ci >= 2)
        def _():
            pltpu.make_async_copy(ov, out_hbm.at[pl.ds(0, QCH)], os).wait()

        def qloop(qi, carry2):
            s0 = wv[pl.ds(qi * 48, 16)]
            s1 = wv[pl.ds(qi * 48 + 16, 16)]
            s2 = wv[pl.ds(qi * 48 + 32, 16)]
            for ch in range(O1 // 16):
                sl = pl.ds(ch * 16, 16)
                ov[qi, sl] = (s0 * rr0[qi, sl] + s1 * rr1[qi, sl]) \
                    + s2 * rr2[qi, sl]
            return carry2

        lax.fori_loop(0, QCH, qloop, 0)
        pltpu.async_copy(ov, out_hbm.at[pl.ds(base + ci * QCH, QCH)], os)

    fire(0, ra0, ra1, ra2, wva, gsa)

    def chunk(ci, carry):
        even = lax.rem(ci, 2) == 0

        @pl.when(even)
        def _():
            process(ci, ra0, ra1, ra2, wva, ova, gsa, osa,
                    rb0, rb1, rb2, wvb, gsb)

        @pl.when(jnp.logical_not(even))
        def _():
            process(ci, rb0, rb1, rb2, wvb, ovb, gsb, osb,
                    ra0, ra1, ra2, wva, gsa)

        return carry

    lax.fori_loop(0, NCH, chunk, 0)
    pltpu.make_async_copy(ova, out_hbm.at[pl.ds(0, QCH)], osa).wait()
    pltpu.make_async_copy(ovb, out_hbm.at[pl.ds(0, QCH)], osb).wait()


def _mlp1_body(y1a_ref, f2_ref, w1b_ref, b1_ref, y_ref, s_ref, q_ref):
    y = y1a_ref[...] + lax.dot_general(
        f2_ref[0].astype(jnp.bfloat16), w1b_ref[...].astype(jnp.bfloat16),
        (((0,), (0,)), ((), ())), preferred_element_type=jnp.float32)
    y = y + b1_ref[...]
    y_ref[...] = y.astype(jnp.bfloat16)

    @pl.when((pl.program_id(0) == 0) & (pl.program_id(1) == 0))
    def _():
        s_ref[...] = jnp.zeros_like(s_ref)
        q_ref[...] = jnp.zeros_like(q_ref)

    s_ref[...] += jnp.sum(y, axis=0, keepdims=True)
    q_ref[...] += jnp.sum(y * y, axis=0, keepdims=True)


def _mlp2_body(y1_ref, s1a_ref, q1a_ref, s1b_ref, q1b_ref, g1_ref, bt1_ref,
               w2_ref, b2_ref, y2_ref, s_ref, q_ref):
    mean1 = (s1a_ref[...] + s1b_ref[...]) * (1.0 / NQ)
    var1 = (q1a_ref[...] + q1b_ref[...]) * (1.0 / NQ) - mean1 * mean1
    a1 = g1_ref[...] / jnp.sqrt(var1 + EPS)
    c1 = bt1_ref[...] - mean1 * a1
    y1 = y1_ref[...].astype(jnp.float32)
    h = jnp.maximum(y1 * a1 + c1, 0.0)
    y = jnp.dot(h.astype(jnp.bfloat16), w2_ref[...].astype(jnp.bfloat16),
                preferred_element_type=jnp.float32)
    y = y + b2_ref[...]
    y2_ref[...] = y.astype(jnp.bfloat16)

    @pl.when(pl.program_id(0) == 0)
    def _():
        s_ref[...] = jnp.zeros_like(s_ref)
        q_ref[...] = jnp.zeros_like(q_ref)

    s_ref[...] += jnp.sum(y, axis=0, keepdims=True)
    q_ref[...] += jnp.sum(y * y, axis=0, keepdims=True)


def _fin_body(y2_ref, s2a_ref, q2a_ref, s2b_ref, q2b_ref, g2_ref, bt2_ref,
              o_ref):
    mean2 = (s2a_ref[...] + s2b_ref[...]) * (1.0 / NQ)
    var2 = (q2a_ref[...] + q2b_ref[...]) * (1.0 / NQ) - mean2 * mean2
    a2 = g2_ref[...] / jnp.sqrt(var2 + EPS)
    c2 = bt2_ref[...] - mean2 * a2
    y2 = y2_ref[...].astype(jnp.float32)
    h = jnp.maximum(y2 * a2 + c2, 0.0)
    o_ref[0] = h.T


def _fin_body_b(prev_ref, y2_ref, s2a_ref, q2a_ref, s2b_ref, q2b_ref,
                g2_ref, bt2_ref, o_ref):
    del prev_ref
    _fin_body(y2_ref, s2a_ref, q2a_ref, s2b_ref, q2b_ref, g2_ref, bt2_ref,
              o_ref)


def _nn3_half(p2t, points1, features1, w1aT, boff):
    return pl.pallas_call(
        functools.partial(_nn3_body, boff=boff),
        grid=(BH, N2 // QT),
        in_specs=[
            pl.BlockSpec((1, QT, 3), lambda bb, t, bo=boff: (bb + bo, t, 0)),
            pl.BlockSpec((1, 3, N1), lambda bb, t, bo=boff: (bb + bo, 0, 0)),
            pl.BlockSpec((1, C1, N1), lambda bb, t, bo=boff: (bb + bo, 0, 0)),
            pl.BlockSpec((C1, O1), lambda bb, t: (0, 0)),
        ],
        out_specs=(
            [pl.BlockSpec((1, QT, 1), lambda bb, t: (bb, t, 0))] * 3
            + [pl.BlockSpec((1, QT, 48), lambda bb, t: (bb, t, 0))]
            + [pl.BlockSpec((N1, O1), lambda bb, t, bo=boff: (bb + bo, 0))]),
        out_shape=(
            [jax.ShapeDtypeStruct((BH, N2, 1), jnp.int32)] * 3
            + [jax.ShapeDtypeStruct((BH, N2, 48), jnp.float32)]
            + [jax.ShapeDtypeStruct((B * N1, O1), jnp.float32)]),
    )(p2t, points1, features1, w1aT)


def _interp_half(nn):
    i0, i1, i2 = (x.reshape(NQH) for x in nn[:3])
    wexp = nn[3].reshape(NQH * 48)
    g = nn[4]
    mesh = plsc.VectorSubcoreMesh(core_axis_name="c", subcore_axis_name="s")
    interp = functools.partial(
        pl.kernel,
        mesh=mesh,
        out_type=jax.ShapeDtypeStruct((NQH, O1), jnp.float32),
        scratch_types=(
            [pltpu.VMEM((QPW,), jnp.int32)] * 3
            + ([pltpu.VMEM((QCH, O1), jnp.float32)] * 3
               + [pltpu.VMEM((QCH * 48,), jnp.float32)]
               + [pltpu.VMEM((QCH, O1), jnp.float32)]) * 2
            + [pltpu.SemaphoreType.DMA] * 4
        ),
    )(_interp_body)
    return interp(g, i0, i1, i2, wexp)


def _mlp1_half(y1a, features2, w1bT, b1r, boff):
    return pl.pallas_call(
        _mlp1_body,
        grid=(BH, N2 // RT),
        in_specs=[
            pl.BlockSpec((RT, O1), lambda bb, t: (bb * (N2 // RT) + t, 0)),
            pl.BlockSpec((1, C2, RT), lambda bb, t, bo=boff: (bb + bo, 0, t)),
            pl.BlockSpec((C2, O1), lambda bb, t: (0, 0)),
            pl.BlockSpec((1, O1), lambda bb, t: (0, 0)),
        ],
        out_specs=[
            pl.BlockSpec((RT, O1), lambda bb, t: (bb * (N2 // RT) + t, 0)),
            pl.BlockSpec((1, O1), lambda bb, t: (0, 0)),
            pl.BlockSpec((1, O1), lambda bb, t: (0, 0)),
        ],
        out_shape=[
            jax.ShapeDtypeStruct((NQH, O1), jnp.bfloat16),
            jax.ShapeDtypeStruct((1, O1), jnp.float32),
            jax.ShapeDtypeStruct((1, O1), jnp.float32),
        ],
    )(y1a, features2, w1bT, b1r)


def _mlp2_half(y1raw, stats1, g1r, bt1r, w2T, b2r):
    return pl.pallas_call(
        _mlp2_body,
        grid=(NQH // RT,),
        in_specs=[
            pl.BlockSpec((RT, O1), lambda i: (i, 0)),
            pl.BlockSpec((1, O1), lambda i: (0, 0)),
            pl.BlockSpec((1, O1), lambda i: (0, 0)),
            pl.BlockSpec((1, O1), lambda i: (0, 0)),
            pl.BlockSpec((1, O1), lambda i: (0, 0)),
            pl.BlockSpec((1, O1), lambda i: (0, 0)),
            pl.BlockSpec((1, O1), lambda i: (0, 0)),
            pl.BlockSpec((O1, O2), lambda i: (0, 0)),
            pl.BlockSpec((1, O2), lambda i: (0, 0)),
        ],
        out_specs=[
            pl.BlockSpec((RT, O2), lambda i: (i, 0)),
            pl.BlockSpec((1, O2), lambda i: (0, 0)),
            pl.BlockSpec((1, O2), lambda i: (0, 0)),
        ],
        out_shape=[
            jax.ShapeDtypeStruct((NQH, O2), jnp.bfloat16),
            jax.ShapeDtypeStruct((1, O2), jnp.float32),
            jax.ShapeDtypeStruct((1, O2), jnp.float32),
        ],
    )(y1raw, *stats1, g1r, bt1r, w2T, b2r)


_FIN_STATS_SPECS = [pl.BlockSpec((1, O2), lambda bb, t: (0, 0))] * 6


def _fin_half_a(y2raw, stats2, g2r, bt2r):
    return pl.pallas_call(
        _fin_body,
        grid=(BH, N2 // NT2),
        in_specs=[
            pl.BlockSpec((NT2, O2), lambda bb, t: (bb * (N2 // NT2) + t, 0)),
        ] + _FIN_STATS_SPECS,
        out_specs=pl.BlockSpec((1, O2, NT2), lambda bb, t: (bb, 0, t)),
        out_shape=jax.ShapeDtypeStruct((B, O2, N2), jnp.float32),
    )(y2raw, *stats2, g2r, bt2r)


def _fin_half_b(out_a, y2raw, stats2, g2r, bt2r):
    return pl.pallas_call(
        _fin_body_b,
        grid=(BH, N2 // NT2),
        in_specs=[
            pl.BlockSpec((1, O2, NT2), lambda bb, t: (0, 0, 0)),
            pl.BlockSpec((NT2, O2), lambda bb, t: (bb * (N2 // NT2) + t, 0)),
        ] + _FIN_STATS_SPECS,
        out_specs=pl.BlockSpec((1, O2, NT2), lambda bb, t: (bb + BH, 0, t)),
        out_shape=jax.ShapeDtypeStruct((B, O2, N2), jnp.float32),
        input_output_aliases={0: 0},
    )(out_a, y2raw, *stats2, g2r, bt2r)


def kernel(points1, points2, features1, features2, W1, b1, gamma1, beta1,
           W2, b2, gamma2, beta2):
    p2t = jnp.transpose(points2, (0, 2, 1))
    w1aT = jnp.transpose(W1[:, :C1])
    w1bT = jnp.transpose(W1[:, C1:])
    w2T = jnp.transpose(W2)
    b1r = b1.reshape(1, O1)
    b2r = b2.reshape(1, O2)
    g1r = gamma1.reshape(1, O1)
    bt1r = beta1.reshape(1, O1)
    g2r = gamma2.reshape(1, O2)
    bt2r = beta2.reshape(1, O2)

    nn_a = _nn3_half(p2t, points1, features1, w1aT, 0)
    y1a_a = _interp_half(nn_a)
    nn_b = _nn3_half(p2t, points1, features1, w1aT, BH)
    y1a_b = _interp_half(nn_b)

    y1raw_a, s1a, q1a = _mlp1_half(y1a_a, features2, w1bT, b1r, 0)
    y1raw_b, s1b, q1b = _mlp1_half(y1a_b, features2, w1bT, b1r, BH)

    stats1 = (s1a, q1a, s1b, q1b)
    y2raw_a, s2a, q2a = _mlp2_half(y1raw_a, stats1, g1r, bt1r, w2T, b2r)
    y2raw_b, s2b, q2b = _mlp2_half(y1raw_b, stats1, g1r, bt1r, w2T, b2r)

    stats2 = (s2a, q2a, s2b, q2b)
    out_a = _fin_half_a(y2raw_a, stats2, g2r, bt2r)
    return _fin_half_b(out_a, y2raw_b, stats2, g2r, bt2r)
